# Initial kernel scaffold; baseline (speedup 1.0000x reference)
import jax
import jax.numpy as jnp
from jax import lax
from jax.experimental import pallas as pl
from jax.experimental.pallas import tpu as pltpu

H = W = 384
NPLANES = 24
_PERMS = ((0, 1, 2, 3), (2, 0, 3, 1), (3, 2, 1, 0), (1, 3, 0, 2))


def _delta(k, off):
    dy, dx = off
    return ((dy, dx), (dx, -dy), (-dy, -dx), (-dx, dy))[k]


_MSB_OFFS = ((0, 1), (1, 1), (1, 2))
_LSB_OFFS = ((0, 1), (1, 1))
MSB_TERMS = tuple(_delta(k, off) for off in _MSB_OFFS for k in range(4))
LSB_TERMS = tuple(_delta(k, off) for off in _LSB_OFFS for k in range(4))


def _pack_tables(luts, scale):
    rows = []
    for lut in luts:
        for k in range(4):
            eff = lut[:, list(_PERMS[k])] * scale
            b = eff.astype(jnp.bfloat16)
            u = lax.bitcast_convert_type(b, jnp.uint16).astype(jnp.uint32)
            p01 = (u[:, 0] << 16) | u[:, 1]
            p23 = (u[:, 2] << 16) | u[:, 3]
            rows += [p01[:128], p01[128:], p23[:128], p23[128:]]
    return jnp.stack(rows)


def _hklut_kernel(img_ref, tm_ref, tl_ref, out_ref, cmp_ref, clp_ref):
    x_plane = img_ref[0]
    xi = (x_plane * 255.0).astype(jnp.int32)
    cm = xi >> 4
    cl = xi & 15

    def padcols(a):
        return jnp.concatenate([a[:, :1], a[:, :1], a, a[:, -1:], a[:, -1:]],
                               axis=1)

    for ref, plane in ((cmp_ref, cm), (clp_ref, cl)):
        ref[8:392, 2:386] = plane
        ref[6:8, :] = padcols(jnp.broadcast_to(plane[:1], (2, W)))
        ref[392:394, :] = padcols(jnp.broadcast_to(plane[-1:], (2, W)))
        ref[8:392, 0:2] = jnp.broadcast_to(plane[:, :1], (H, 2))
        ref[8:392, 386:388] = jnp.broadcast_to(plane[:, -1:], (H, 2))

    lane = lax.broadcasted_iota(jnp.int32, (8, 128), 1)
    sub = lax.broadcasted_iota(jnp.int32, (8, 128), 0)
    lane_par = (lane & 1) == 1
    ilane = lane >> 1
    ilane_r = ilane + 64
    sub_par = (sub & 1) == 1
    isub_t = sub >> 1
    isub_b = isub_t + 4

    def gather_l(tbl, idx):
        return jnp.take_along_axis(tbl, idx, axis=1, mode="promise_in_bounds")

    def gather_s(arr, idx):
        return jnp.take_along_axis(arr, idx, axis=0, mode="promise_in_bounds")

    def strip(i, carry):
        r0 = i * 8
        for c0 in (0, 128, 256):
            x = img_ref[0, pl.ds(r0, 8), pl.ds(c0, 128)]
            xi_t = (x * 255.0).astype(jnp.int32)
            cm_c = xi_t >> 4
            cl_c = xi_t & 15
            mm = cm_c >= 8
            ml = cl_c >= 8
            cm716 = (cm_c & 7) << 4
            cl716 = (cl_c & 7) << 4
            acc = [x, x, x, x]

            for terms, tref, pad_ref, base16, msk in (
                (MSB_TERMS, tm_ref, cmp_ref, cm716, mm),
                (LSB_TERMS, tl_ref, clp_ref, cl716, ml),
            ):
                for t, (dy, dx) in enumerate(terms):
                    nb = pad_ref[pl.ds(8 + r0 + dy, 8), pl.ds(2 + c0 + dx, 128)]
                    idx7 = base16 | nb
                    for pair in (0, 1):
                        tlo = jnp.broadcast_to(
                            tref[pl.ds(4 * t + 2 * pair, 1), :], (8, 128))
                        thi = jnp.broadcast_to(
                            tref[pl.ds(4 * t + 2 * pair + 1, 1), :], (8, 128))
                        g = jnp.where(msk, gather_l(thi, idx7),
                                      gather_l(tlo, idx7))
                        acc[2 * pair] += lax.bitcast_convert_type(
                            g & jnp.uint32(0xFFFF0000), jnp.float32)
                        acc[2 * pair + 1] += lax.bitcast_convert_type(
                            g << 16, jnp.float32)

            acc = [jnp.clip(a, 0.0, 1.0) for a in acc]
            el = jnp.where(lane_par, gather_l(acc[1], ilane),
                           gather_l(acc[0], ilane))
            er = jnp.where(lane_par, gather_l(acc[1], ilane_r),
                           gather_l(acc[0], ilane_r))
            ol = jnp.where(lane_par, gather_l(acc[3], ilane),
                           gather_l(acc[2], ilane))
            orr = jnp.where(lane_par, gather_l(acc[3], ilane_r),
                            gather_l(acc[2], ilane_r))
            tlv = jnp.where(sub_par, gather_s(ol, isub_t), gather_s(el, isub_t))
            trv = jnp.where(sub_par, gather_s(orr, isub_t), gather_s(er, isub_t))
            blv = jnp.where(sub_par, gather_s(ol, isub_b), gather_s(el, isub_b))
            brv = jnp.where(sub_par, gather_s(orr, isub_b), gather_s(er, isub_b))
            out_ref[0, pl.ds(2 * r0, 8), pl.ds(2 * c0, 128)] = tlv
            out_ref[0, pl.ds(2 * r0, 8), pl.ds(2 * c0 + 128, 128)] = trv
            out_ref[0, pl.ds(2 * r0 + 8, 8), pl.ds(2 * c0, 128)] = blv
            out_ref[0, pl.ds(2 * r0 + 8, 8), pl.ds(2 * c0 + 128, 128)] = brv
        return carry

    lax.fori_loop(0, H // 8, strip, 0)


@jax.jit
def kernel(img_lr, h_msb, d_msb, b_msb, h_lsb, d_lsb):
    B, C = img_lr.shape[0], img_lr.shape[1]
    tm = _pack_tables((h_msb, d_msb, b_msb), 1.0 / (12.0 * 255.0))
    tl = _pack_tables((h_lsb, d_lsb), 1.0 / (8.0 * 255.0))
    img = img_lr.reshape(NPLANES, H, W)

    out = pl.pallas_call(
        _hklut_kernel,
        grid=(NPLANES,),
        in_specs=[
            pl.BlockSpec((1, H, W), lambda p: (p, 0, 0)),
            pl.BlockSpec((4 * len(MSB_TERMS), 128), lambda p: (0, 0)),
            pl.BlockSpec((4 * len(LSB_TERMS), 128), lambda p: (0, 0)),
        ],
        out_specs=pl.BlockSpec((1, 2 * H, 2 * W), lambda p: (p, 0, 0)),
        out_shape=jax.ShapeDtypeStruct((NPLANES, 2 * H, 2 * W), jnp.float32),
        scratch_shapes=[
            pltpu.VMEM((400, 388), jnp.int32),
            pltpu.VMEM((400, 388), jnp.int32),
        ],
        compiler_params=pltpu.CompilerParams(
            dimension_semantics=("arbitrary",),
        ),
    )(img, tm, tl)
    return out.reshape(B, C, 2 * H, 2 * W)


# speedup vs baseline: 93.7102x; 93.7102x over previous
import jax
import jax.numpy as jnp
from jax import lax
from jax.experimental import pallas as pl
from jax.experimental.pallas import tpu as pltpu

H = W = 384
NPLANES = 24
_PERMS = ((0, 1, 2, 3), (2, 0, 3, 1), (3, 2, 1, 0), (1, 3, 0, 2))


def _delta(k, off):
    dy, dx = off
    return ((dy, dx), (dx, -dy), (-dy, -dx), (-dx, dy))[k]


_MSB_OFFS = ((0, 1), (1, 1), (1, 2))
_LSB_OFFS = ((0, 1), (1, 1))
MSB_TERMS = tuple(_delta(k, off) for off in _MSB_OFFS for k in range(4))
LSB_TERMS = tuple(_delta(k, off) for off in _LSB_OFFS for k in range(4))


def _pack_tables(luts, scale):
    rows = []
    for lut in luts:
        for k in range(4):
            eff = lut[:, list(_PERMS[k])] * scale
            b = eff.astype(jnp.bfloat16)
            u = lax.bitcast_convert_type(b, jnp.uint16).astype(jnp.uint32)
            p01 = (u[:, 0] << 16) | u[:, 1]
            p23 = (u[:, 2] << 16) | u[:, 3]
            rows += [p01[:128], p01[128:], p23[:128], p23[128:]]
    return jnp.stack(rows)


_CM_DYS = (-2, -1, 0, 1, 2)
_CL_DYS = (-1, 0, 1)


def _hklut_kernel(img_ref, tm_ref, tl_ref, out_ref, cmp_ref, clp_ref):
    x_plane = img_ref[0]
    xi = (x_plane * 255.0).astype(jnp.int32)
    cm = xi >> 4
    cl = xi & 15

    def padcols(a):
        return jnp.concatenate([a[:, :1], a[:, :1], a, a[:, -1:], a[:, -1:]],
                               axis=1)

    def rowshift(a, dy):
        if dy == 0:
            return a
        if dy > 0:
            return jnp.concatenate([a[dy:]] + dy * [a[-1:]], axis=0)
        return jnp.concatenate((-dy) * [a[:1]] + [a[:dy]], axis=0)

    for ref, plane, dys in ((cmp_ref, cm, _CM_DYS), (clp_ref, cl, _CL_DYS)):
        for j, dy in enumerate(dys):
            ref[j] = padcols(rowshift(plane, dy))

    lane = lax.broadcasted_iota(jnp.int32, (8, 128), 1)
    sub = lax.broadcasted_iota(jnp.int32, (8, 128), 0)
    lane_par = (lane & 1) == 1
    ilane = lane >> 1
    ilane_r = ilane + 64
    sub_par = (sub & 1) == 1
    isub_t = sub >> 1
    isub_b = isub_t + 4

    def gather_l(tbl, idx):
        return jnp.take_along_axis(tbl, idx, axis=1, mode="promise_in_bounds")

    def gather_s(arr, idx):
        return jnp.take_along_axis(arr, idx, axis=0, mode="promise_in_bounds")

    def strip(i, carry):
        r0 = i * 8
        for c0 in (0, 128, 256):
            x = img_ref[0, pl.ds(r0, 8), pl.ds(c0, 128)]
            xi_t = (x * 255.0).astype(jnp.int32)
            cm_c = xi_t >> 4
            cl_c = xi_t & 15
            mm = cm_c >= 8
            ml = cl_c >= 8
            cm716 = (cm_c & 7) << 4
            cl716 = (cl_c & 7) << 4
            acc = [x, x, x, x]

            for terms, tref, pad_ref, dys, base16, msk in (
                (MSB_TERMS, tm_ref, cmp_ref, _CM_DYS, cm716, mm),
                (LSB_TERMS, tl_ref, clp_ref, _CL_DYS, cl716, ml),
            ):
                for t, (dy, dx) in enumerate(terms):
                    nb = pad_ref[dys.index(dy), pl.ds(r0, 8),
                                 pl.ds(2 + c0 + dx, 128)]
                    idx7 = base16 | nb
                    for pair in (0, 1):
                        tlo = jnp.broadcast_to(
                            tref[pl.ds(4 * t + 2 * pair, 1), :], (8, 128))
                        thi = jnp.broadcast_to(
                            tref[pl.ds(4 * t + 2 * pair + 1, 1), :], (8, 128))
                        g = jnp.where(msk, gather_l(thi, idx7),
                                      gather_l(tlo, idx7))
                        acc[2 * pair] += lax.bitcast_convert_type(
                            g & jnp.uint32(0xFFFF0000), jnp.float32)
                        acc[2 * pair + 1] += lax.bitcast_convert_type(
                            g << 16, jnp.float32)

            acc = [jnp.clip(a, 0.0, 1.0) for a in acc]
            el = jnp.where(lane_par, gather_l(acc[1], ilane),
                           gather_l(acc[0], ilane))
            er = jnp.where(lane_par, gather_l(acc[1], ilane_r),
                           gather_l(acc[0], ilane_r))
            ol = jnp.where(lane_par, gather_l(acc[3], ilane),
                           gather_l(acc[2], ilane))
            orr = jnp.where(lane_par, gather_l(acc[3], ilane_r),
                            gather_l(acc[2], ilane_r))
            tlv = jnp.where(sub_par, gather_s(ol, isub_t), gather_s(el, isub_t))
            trv = jnp.where(sub_par, gather_s(orr, isub_t), gather_s(er, isub_t))
            blv = jnp.where(sub_par, gather_s(ol, isub_b), gather_s(el, isub_b))
            brv = jnp.where(sub_par, gather_s(orr, isub_b), gather_s(er, isub_b))
            out_ref[0, pl.ds(2 * r0, 8), pl.ds(2 * c0, 128)] = tlv
            out_ref[0, pl.ds(2 * r0, 8), pl.ds(2 * c0 + 128, 128)] = trv
            out_ref[0, pl.ds(2 * r0 + 8, 8), pl.ds(2 * c0, 128)] = blv
            out_ref[0, pl.ds(2 * r0 + 8, 8), pl.ds(2 * c0 + 128, 128)] = brv
        return carry

    lax.fori_loop(0, H // 8, strip, 0)


@jax.jit
def kernel(img_lr, h_msb, d_msb, b_msb, h_lsb, d_lsb):
    B, C = img_lr.shape[0], img_lr.shape[1]
    tm = _pack_tables((h_msb, d_msb, b_msb), 1.0 / (12.0 * 255.0))
    tl = _pack_tables((h_lsb, d_lsb), 1.0 / (8.0 * 255.0))
    img = img_lr.reshape(NPLANES, H, W)

    out = pl.pallas_call(
        _hklut_kernel,
        grid=(NPLANES,),
        in_specs=[
            pl.BlockSpec((1, H, W), lambda p: (p, 0, 0)),
            pl.BlockSpec((4 * len(MSB_TERMS), 128), lambda p: (0, 0)),
            pl.BlockSpec((4 * len(LSB_TERMS), 128), lambda p: (0, 0)),
        ],
        out_specs=pl.BlockSpec((1, 2 * H, 2 * W), lambda p: (p, 0, 0)),
        out_shape=jax.ShapeDtypeStruct((NPLANES, 2 * H, 2 * W), jnp.float32),
        scratch_shapes=[
            pltpu.VMEM((len(_CM_DYS), H, W + 4), jnp.int32),
            pltpu.VMEM((len(_CL_DYS), H, W + 4), jnp.int32),
        ],
        compiler_params=pltpu.CompilerParams(
            dimension_semantics=("arbitrary",),
        ),
    )(img, tm, tl)
    return out.reshape(B, C, 2 * H, 2 * W)


# speedup vs baseline: 184.6893x; 1.9709x over previous
import dataclasses
import functools

import jax
import jax.numpy as jnp
from jax import lax
from jax.experimental import pallas as pl
from jax.experimental.pallas import tpu as pltpu
from jax.experimental.pallas import tpu_sc as plsc

H = W = 384
NPLANES = 24
N_SC = 12
_PERMS = ((0, 1, 2, 3), (2, 0, 3, 1), (3, 2, 1, 0), (1, 3, 0, 2))


def _delta(k, off):
    dy, dx = off
    return ((dy, dx), (dx, -dy), (-dy, -dx), (-dx, dy))[k]


_MSB_OFFS = ((0, 1), (1, 1), (1, 2))
_LSB_OFFS = ((0, 1), (1, 1))
MSB_TERMS = tuple(_delta(k, off) for off in _MSB_OFFS for k in range(4))
LSB_TERMS = tuple(_delta(k, off) for off in _LSB_OFFS for k in range(4))


def _pack_tables(luts, scale):
    rows = []
    for lut in luts:
        for k in range(4):
            eff = lut[:, list(_PERMS[k])] * scale
            b = eff.astype(jnp.bfloat16)
            u = lax.bitcast_convert_type(b, jnp.uint16).astype(jnp.uint32)
            p01 = (u[:, 0] << 16) | u[:, 1]
            p23 = (u[:, 2] << 16) | u[:, 3]
            rows += [p01[:128], p01[128:], p23[:128], p23[128:]]
    return jnp.stack(rows)


_CM_DYS = (-2, -1, 0, 1, 2)
_CL_DYS = (-1, 0, 1)

_DIR_MAP = {}
for _ti, _d in enumerate(MSB_TERMS):
    _DIR_MAP[_d] = [_ti, None]
for _li, _d in enumerate(LSB_TERMS):
    _DIR_MAP[_d][1] = _li
_DIRS = tuple((dy, dx, v[0], v[1]) for (dy, dx), v in _DIR_MAP.items())


def _sc_tables(h_msb, d_msb, b_msb, h_lsb, d_lsb):
    rows = []
    for luts, scale in (((h_msb, d_msb, b_msb), 1.0 / (12.0 * 255.0)),
                        ((h_lsb, d_lsb), 1.0 / (8.0 * 255.0))):
        for lut in luts:
            for k in range(4):
                rows.append((lut[:, list(_PERMS[k])] * scale).reshape(-1))
    return jnp.stack(rows)


def _sc_compiler_params():
    cp = pltpu.CompilerParams()
    if "needs_layout_passes" in pltpu.CompilerParams.__dataclass_fields__:
        cp = dataclasses.replace(cp, needs_layout_passes=False)
    return cp


def _sc_hklut(img_sc, tbl):
    n_sc = img_sc.shape[0]
    info = plsc.get_sparse_core_info()
    nc = info.num_cores
    mesh = plsc.VectorSubcoreMesh(core_axis_name="c", subcore_axis_name="s")

    @functools.partial(
        pl.kernel,
        out_type=jax.ShapeDtypeStruct((n_sc, 2 * H, 2 * W), jnp.float32),
        mesh=mesh,
        scratch_types=[
            pltpu.VMEM((24, W), jnp.float32),
            pltpu.VMEM((24, W), jnp.int32),
            pltpu.VMEM((24, 2 * W), jnp.float32),
            pltpu.VMEM((20, 1024), jnp.float32),
        ],
        compiler_params=_sc_compiler_params(),
    )
    def k(img_hbm, tbl_hbm, out_hbm, imgc, xic, outc, tblv):
        wid = lax.axis_index("s") * nc + lax.axis_index("c")
        r0 = wid * 12
        base = pl.multiple_of(jnp.clip(((r0 - 2) // 8) * 8, 0, H - 24), 8)
        off = r0 - base
        iota = lax.iota(jnp.int32, 16)
        iota2 = iota * 2
        zero = jnp.zeros((16,), jnp.int32)
        pltpu.sync_copy(tbl_hbm, tblv)

        @pl.loop(0, n_sc)
        def _plane(p):
            pltpu.sync_copy(img_hbm.at[p, pl.ds(base, 24)], imgc)

            @pl.loop(0, 24)
            def _r(r):
                rv = zero + r

                @pl.loop(0, W // 16)
                def _c(cc):
                    cv = cc * 16 + iota
                    xv = plsc.load_gather(imgc, [rv, cv])
                    xiv = (xv * 255.0).astype(jnp.int32)
                    plsc.store_scatter(xic, [rv, cv], xiv)

            @pl.loop(0, 12)
            def _row(y):
                yy = y + off
                rowvs = {dy: jnp.clip(zero + (yy + dy), 0, 23)
                         for dy in _CM_DYS}
                rowe = zero + 2 * y
                rowo = rowe + 1

                @pl.loop(0, W // 16)
                def _x(xx):
                    xb = xx * 16
                    colvs = {dx: jnp.clip(xb + iota + dx, 0, W - 1)
                             for dx in _CM_DYS}
                    rv0, cv0 = rowvs[0], colvs[0]
                    xv = plsc.load_gather(imgc, [rv0, cv0])
                    xiv = plsc.load_gather(xic, [rv0, cv0])
                    cm16 = (xiv >> 4) << 4
                    cl16 = (xiv & 15) << 4
                    accs = [xv, xv, xv, xv]
                    for (dy, dx, mt, lt) in _DIRS:
                        nxi = plsc.load_gather(xic, [rowvs[dy], colvs[dx]])
                        pairs = []
                        if mt is not None:
                            pairs.append((mt, (cm16 + (nxi >> 4)) << 2))
                        if lt is not None:
                            pairs.append((12 + lt, (cl16 + (nxi & 15)) << 2))
                        for trow, i4 in pairs:
                            tv = zero + trow
                            for j in range(4):
                                g = plsc.load_gather(tblv, [tv, i4 + j])
                                accs[j] = accs[j] + g
                    accs = [jnp.clip(a, 0.0, 1.0) for a in accs]
                    ce = xb * 2 + iota2
                    plsc.store_scatter(outc, [rowe, ce], accs[0])
                    plsc.store_scatter(outc, [rowe, ce + 1], accs[1])
                    plsc.store_scatter(outc, [rowo, ce], accs[2])
                    plsc.store_scatter(outc, [rowo, ce + 1], accs[3])

            pltpu.sync_copy(outc, out_hbm.at[p, pl.ds(pl.multiple_of(2 * r0, 8), 24)])

    return k(img_sc, tbl)


def _hklut_kernel(img_ref, tm_ref, tl_ref, out_ref, cmp_ref, clp_ref):
    x_plane = img_ref[0]
    xi = (x_plane * 255.0).astype(jnp.int32)
    cm = xi >> 4
    cl = xi & 15

    def padcols(a):
        return jnp.concatenate([a[:, :1], a[:, :1], a, a[:, -1:], a[:, -1:]],
                               axis=1)

    def rowshift(a, dy):
        if dy == 0:
            return a
        if dy > 0:
            return jnp.concatenate([a[dy:]] + dy * [a[-1:]], axis=0)
        return jnp.concatenate((-dy) * [a[:1]] + [a[:dy]], axis=0)

    for ref, plane, dys in ((cmp_ref, cm, _CM_DYS), (clp_ref, cl, _CL_DYS)):
        for j, dy in enumerate(dys):
            ref[j] = padcols(rowshift(plane, dy))

    def gather_l(tbl, idx):
        return jnp.take_along_axis(tbl, idx, axis=1, mode="promise_in_bounds")

    def gather_s(arr, idx):
        return jnp.take_along_axis(arr, idx, axis=0, mode="promise_in_bounds")

    lane = lax.broadcasted_iota(jnp.int32, (8, 128), 1)
    sub = lax.broadcasted_iota(jnp.int32, (8, 128), 0)
    lane_par = (lane & 1) == 1
    ilane = lane >> 1
    ilane_r = ilane + 64
    sub_par = (sub & 1) == 1
    isub_t = sub >> 1
    isub_b = isub_t + 4

    COLS = (0, 128, 256)

    def strip(i, carry):
        r0 = i * 8
        for c0 in COLS:
            x = img_ref[0, pl.ds(r0, 8), pl.ds(c0, 128)]
            xi_t = (x * 255.0).astype(jnp.int32)
            cm_c = xi_t >> 4
            cl_c = xi_t & 15
            mm = cm_c >= 8
            ml = cl_c >= 8
            cm716 = (cm_c & 7) << 4
            cl716 = (cl_c & 7) << 4
            acc = [x, x, x, x]

            for terms, tref, pad_ref, dys, base16, msk in (
                (MSB_TERMS, tm_ref, cmp_ref, _CM_DYS, cm716, mm),
                (LSB_TERMS, tl_ref, clp_ref, _CL_DYS, cl716, ml),
            ):
                for t, (dy, dx) in enumerate(terms):
                    nb = pad_ref[dys.index(dy), pl.ds(r0, 8),
                                 pl.ds(2 + c0 + dx, 128)]
                    idx7 = base16 | nb
                    for pair in (0, 1):
                        tlo = tref[4 * t + 2 * pair]
                        thi = tref[4 * t + 2 * pair + 1]
                        g = jnp.where(msk, gather_l(thi, idx7),
                                      gather_l(tlo, idx7))
                        acc[2 * pair] += lax.bitcast_convert_type(
                            g & jnp.uint32(0xFFFF0000), jnp.float32)
                        acc[2 * pair + 1] += lax.bitcast_convert_type(
                            g << 16, jnp.float32)

            acc = [jnp.clip(a, 0.0, 1.0) for a in acc]
            el = jnp.where(lane_par, gather_l(acc[1], ilane),
                           gather_l(acc[0], ilane))
            er = jnp.where(lane_par,
                           gather_l(pltpu.roll(acc[1], 64, 1), ilane),
                           gather_l(pltpu.roll(acc[0], 64, 1), ilane))
            ol = jnp.where(lane_par, gather_l(acc[3], ilane),
                           gather_l(acc[2], ilane))
            orr = jnp.where(lane_par,
                            gather_l(pltpu.roll(acc[3], 64, 1), ilane),
                            gather_l(pltpu.roll(acc[2], 64, 1), ilane))
            tlv = jnp.where(sub_par, gather_s(ol, isub_t), gather_s(el, isub_t))
            trv = jnp.where(sub_par, gather_s(orr, isub_t), gather_s(er, isub_t))
            blv = jnp.where(sub_par, gather_s(ol, isub_b), gather_s(el, isub_b))
            brv = jnp.where(sub_par, gather_s(orr, isub_b), gather_s(er, isub_b))
            out_ref[0, pl.ds(2 * r0, 8), pl.ds(2 * c0, 128)] = tlv
            out_ref[0, pl.ds(2 * r0, 8), pl.ds(2 * c0 + 128, 128)] = trv
            out_ref[0, pl.ds(2 * r0 + 8, 8), pl.ds(2 * c0, 128)] = blv
            out_ref[0, pl.ds(2 * r0 + 8, 8), pl.ds(2 * c0 + 128, 128)] = brv
        return carry

    lax.fori_loop(0, H // 8, strip, 0)


@jax.jit
def kernel(img_lr, h_msb, d_msb, b_msb, h_lsb, d_lsb):
    B, C = img_lr.shape[0], img_lr.shape[1]
    tm = _pack_tables((h_msb, d_msb, b_msb), 1.0 / (12.0 * 255.0))
    tl = _pack_tables((h_lsb, d_lsb), 1.0 / (8.0 * 255.0))
    tm = jnp.broadcast_to(tm[:, None, :], (tm.shape[0], 8, 128))
    tl = jnp.broadcast_to(tl[:, None, :], (tl.shape[0], 8, 128))
    img = img_lr.reshape(NPLANES, H, W)
    n_tc = NPLANES - N_SC

    out_sc = _sc_hklut(img[n_tc:],
                       _sc_tables(h_msb, d_msb, b_msb, h_lsb, d_lsb))

    out = pl.pallas_call(
        _hklut_kernel,
        grid=(n_tc,),
        in_specs=[
            pl.BlockSpec((1, H, W), lambda p: (p, 0, 0)),
            pl.BlockSpec((4 * len(MSB_TERMS), 8, 128), lambda p: (0, 0, 0)),
            pl.BlockSpec((4 * len(LSB_TERMS), 8, 128), lambda p: (0, 0, 0)),
        ],
        out_specs=pl.BlockSpec((1, 2 * H, 2 * W), lambda p: (p, 0, 0)),
        out_shape=jax.ShapeDtypeStruct((n_tc, 2 * H, 2 * W), jnp.float32),
        scratch_shapes=[
            pltpu.VMEM((len(_CM_DYS), H, W + 4), jnp.int32),
            pltpu.VMEM((len(_CL_DYS), H, W + 4), jnp.int32),
        ],
        compiler_params=pltpu.CompilerParams(
            dimension_semantics=("arbitrary",),
        ),
    )(img[:n_tc], tm, tl)
    out = jnp.concatenate([out, out_sc], axis=0)
    return out.reshape(B, C, 2 * H, 2 * W)


# speedup vs baseline: 445.3464x; 2.4113x over previous
import dataclasses
import functools

import jax
import jax.numpy as jnp
from jax import lax
from jax.experimental import pallas as pl
from jax.experimental.pallas import tpu as pltpu
from jax.experimental.pallas import tpu_sc as plsc

H = W = 384
NPLANES = 24
N_SC = 20
_PERMS = ((0, 1, 2, 3), (2, 0, 3, 1), (3, 2, 1, 0), (1, 3, 0, 2))


def _delta(k, off):
    dy, dx = off
    return ((dy, dx), (dx, -dy), (-dy, -dx), (-dx, dy))[k]


_MSB_OFFS = ((0, 1), (1, 1), (1, 2))
_LSB_OFFS = ((0, 1), (1, 1))
MSB_TERMS = tuple(_delta(k, off) for off in _MSB_OFFS for k in range(4))
LSB_TERMS = tuple(_delta(k, off) for off in _LSB_OFFS for k in range(4))


def _pack_tables(luts, scale):
    rows = []
    for lut in luts:
        for k in range(4):
            eff = lut[:, list(_PERMS[k])] * scale
            b = eff.astype(jnp.bfloat16)
            u = lax.bitcast_convert_type(b, jnp.uint16).astype(jnp.uint32)
            p01 = (u[:, 0] << 16) | u[:, 1]
            p23 = (u[:, 2] << 16) | u[:, 3]
            rows += [p01[:128], p01[128:], p23[:128], p23[128:]]
    return jnp.stack(rows)


_CM_DYS = (-2, -1, 0, 1, 2)
_CL_DYS = (-1, 0, 1)

_DIR_MAP = {}
for _ti, _d in enumerate(MSB_TERMS):
    _DIR_MAP[_d] = [_ti, None]
for _li, _d in enumerate(LSB_TERMS):
    _DIR_MAP[_d][1] = _li
_DIRS = tuple((dy, dx, v[0], v[1]) for (dy, dx), v in _DIR_MAP.items())


def _sc_tables(h_msb, d_msb, b_msb, h_lsb, d_lsb):
    rows = []
    for luts, scale in (((h_msb, d_msb, b_msb), 1.0 / (12.0 * 255.0)),
                        ((h_lsb, d_lsb), 1.0 / (8.0 * 255.0))):
        for lut in luts:
            for k in range(4):
                rows.append((lut[:, list(_PERMS[k])] * scale).reshape(-1))
    return jnp.stack(rows)


def _sc_compiler_params():
    cp = pltpu.CompilerParams()
    if "needs_layout_passes" in pltpu.CompilerParams.__dataclass_fields__:
        cp = dataclasses.replace(cp, needs_layout_passes=False)
    return cp


def _sc_hklut(img_sc, tbl):
    n_sc = img_sc.shape[0]
    info = plsc.get_sparse_core_info()
    nc = info.num_cores
    mesh = plsc.VectorSubcoreMesh(core_axis_name="c", subcore_axis_name="s")

    @functools.partial(
        pl.kernel,
        out_type=jax.ShapeDtypeStruct((n_sc, 2 * H, 2 * W), jnp.float32),
        mesh=mesh,
        scratch_types=[
            pltpu.VMEM((24, W), jnp.float32),
            pltpu.VMEM((24, W), jnp.int32),
            pltpu.VMEM((24, 2 * W), jnp.float32),
            pltpu.VMEM((20, 1024), jnp.float32),
        ],
        compiler_params=_sc_compiler_params(),
    )
    def k(img_hbm, tbl_hbm, out_hbm, imgc, xic, outc, tblv):
        wid = lax.axis_index("s") * nc + lax.axis_index("c")
        r0 = wid * 12
        base = pl.multiple_of(jnp.clip(((r0 - 2) // 8) * 8, 0, H - 24), 8)
        off = r0 - base
        iota = lax.iota(jnp.int32, 16)
        iota2 = iota * 2
        zero = jnp.zeros((16,), jnp.int32)
        pltpu.sync_copy(tbl_hbm, tblv)

        @pl.loop(0, n_sc)
        def _plane(p):
            pltpu.sync_copy(img_hbm.at[p, pl.ds(base, 24)], imgc)

            @pl.loop(0, 24)
            def _r(r):
                rv = zero + r

                @pl.loop(0, W // 16)
                def _c(cc):
                    cv = cc * 16 + iota
                    xv = plsc.load_gather(imgc, [rv, cv])
                    xiv = (xv * 255.0).astype(jnp.int32)
                    plsc.store_scatter(xic, [rv, cv], xiv)

            @pl.loop(0, 12)
            def _row(y):
                yy = y + off
                rowvs = {dy: jnp.clip(zero + (yy + dy), 0, 23)
                         for dy in _CM_DYS}
                rowe = zero + 2 * y
                rowo = rowe + 1

                @pl.loop(0, W // 16)
                def _x(xx):
                    xb = xx * 16
                    colvs = {dx: jnp.clip(xb + iota + dx, 0, W - 1)
                             for dx in _CM_DYS}
                    rv0, cv0 = rowvs[0], colvs[0]
                    xv = plsc.load_gather(imgc, [rv0, cv0])
                    xiv = plsc.load_gather(xic, [rv0, cv0])
                    cm16 = (xiv >> 4) << 4
                    cl16 = (xiv & 15) << 4
                    accs = [xv, xv, xv, xv]
                    for (dy, dx, mt, lt) in _DIRS:
                        nxi = plsc.load_gather(xic, [rowvs[dy], colvs[dx]])
                        pairs = []
                        if mt is not None:
                            pairs.append((mt, (cm16 + (nxi >> 4)) << 2))
                        if lt is not None:
                            pairs.append((12 + lt, (cl16 + (nxi & 15)) << 2))
                        for trow, i4 in pairs:
                            tv = zero + trow
                            for j in range(4):
                                g = plsc.load_gather(tblv, [tv, i4 + j])
                                accs[j] = accs[j] + g
                    accs = [jnp.clip(a, 0.0, 1.0) for a in accs]
                    ce = xb * 2 + iota2
                    plsc.store_scatter(outc, [rowe, ce], accs[0])
                    plsc.store_scatter(outc, [rowe, ce + 1], accs[1])
                    plsc.store_scatter(outc, [rowo, ce], accs[2])
                    plsc.store_scatter(outc, [rowo, ce + 1], accs[3])

            pltpu.sync_copy(outc, out_hbm.at[p, pl.ds(pl.multiple_of(2 * r0, 8), 24)])

    return k(img_sc, tbl)


def _hklut_kernel(img_ref, tm_ref, tl_ref, out_ref, cmp_ref, clp_ref):
    x_plane = img_ref[0]
    xi = (x_plane * 255.0).astype(jnp.int32)
    cm = xi >> 4
    cl = xi & 15

    def padcols(a):
        return jnp.concatenate([a[:, :1], a[:, :1], a, a[:, -1:], a[:, -1:]],
                               axis=1)

    def rowshift(a, dy):
        if dy == 0:
            return a
        if dy > 0:
            return jnp.concatenate([a[dy:]] + dy * [a[-1:]], axis=0)
        return jnp.concatenate((-dy) * [a[:1]] + [a[:dy]], axis=0)

    for ref, plane, dys in ((cmp_ref, cm, _CM_DYS), (clp_ref, cl, _CL_DYS)):
        for j, dy in enumerate(dys):
            ref[j] = padcols(rowshift(plane, dy))

    def gather_l(tbl, idx):
        return jnp.take_along_axis(tbl, idx, axis=1, mode="promise_in_bounds")

    def gather_s(arr, idx):
        return jnp.take_along_axis(arr, idx, axis=0, mode="promise_in_bounds")

    lane = lax.broadcasted_iota(jnp.int32, (8, 128), 1)
    sub = lax.broadcasted_iota(jnp.int32, (8, 128), 0)
    lane_par = (lane & 1) == 1
    ilane = lane >> 1
    ilane_r = ilane + 64
    sub_par = (sub & 1) == 1
    isub_t = sub >> 1
    isub_b = isub_t + 4

    COLS = (0, 128, 256)

    def strip(i, carry):
        r0 = i * 8
        for c0 in COLS:
            x = img_ref[0, pl.ds(r0, 8), pl.ds(c0, 128)]
            xi_t = (x * 255.0).astype(jnp.int32)
            cm_c = xi_t >> 4
            cl_c = xi_t & 15
            mm = cm_c >= 8
            ml = cl_c >= 8
            cm716 = (cm_c & 7) << 4
            cl716 = (cl_c & 7) << 4
            acc = [x, x, x, x]

            for terms, tref, pad_ref, dys, base16, msk in (
                (MSB_TERMS, tm_ref, cmp_ref, _CM_DYS, cm716, mm),
                (LSB_TERMS, tl_ref, clp_ref, _CL_DYS, cl716, ml),
            ):
                for t, (dy, dx) in enumerate(terms):
                    nb = pad_ref[dys.index(dy), pl.ds(r0, 8),
                                 pl.ds(2 + c0 + dx, 128)]
                    idx7 = base16 | nb
                    for pair in (0, 1):
                        tlo = tref[4 * t + 2 * pair]
                        thi = tref[4 * t + 2 * pair + 1]
                        g = jnp.where(msk, gather_l(thi, idx7),
                                      gather_l(tlo, idx7))
                        acc[2 * pair] += lax.bitcast_convert_type(
                            g & jnp.uint32(0xFFFF0000), jnp.float32)
                        acc[2 * pair + 1] += lax.bitcast_convert_type(
                            g << 16, jnp.float32)

            acc = [jnp.clip(a, 0.0, 1.0) for a in acc]
            el = jnp.where(lane_par, gather_l(acc[1], ilane),
                           gather_l(acc[0], ilane))
            er = jnp.where(lane_par,
                           gather_l(pltpu.roll(acc[1], 64, 1), ilane),
                           gather_l(pltpu.roll(acc[0], 64, 1), ilane))
            ol = jnp.where(lane_par, gather_l(acc[3], ilane),
                           gather_l(acc[2], ilane))
            orr = jnp.where(lane_par,
                            gather_l(pltpu.roll(acc[3], 64, 1), ilane),
                            gather_l(pltpu.roll(acc[2], 64, 1), ilane))
            tlv = jnp.where(sub_par, gather_s(ol, isub_t), gather_s(el, isub_t))
            trv = jnp.where(sub_par, gather_s(orr, isub_t), gather_s(er, isub_t))
            blv = jnp.where(sub_par, gather_s(ol, isub_b), gather_s(el, isub_b))
            brv = jnp.where(sub_par, gather_s(orr, isub_b), gather_s(er, isub_b))
            out_ref[0, pl.ds(2 * r0, 8), pl.ds(2 * c0, 128)] = tlv
            out_ref[0, pl.ds(2 * r0, 8), pl.ds(2 * c0 + 128, 128)] = trv
            out_ref[0, pl.ds(2 * r0 + 8, 8), pl.ds(2 * c0, 128)] = blv
            out_ref[0, pl.ds(2 * r0 + 8, 8), pl.ds(2 * c0 + 128, 128)] = brv
        return carry

    lax.fori_loop(0, H // 8, strip, 0)


@jax.jit
def kernel(img_lr, h_msb, d_msb, b_msb, h_lsb, d_lsb):
    B, C = img_lr.shape[0], img_lr.shape[1]
    tm = _pack_tables((h_msb, d_msb, b_msb), 1.0 / (12.0 * 255.0))
    tl = _pack_tables((h_lsb, d_lsb), 1.0 / (8.0 * 255.0))
    tm = jnp.broadcast_to(tm[:, None, :], (tm.shape[0], 8, 128))
    tl = jnp.broadcast_to(tl[:, None, :], (tl.shape[0], 8, 128))
    img = img_lr.reshape(NPLANES, H, W)
    n_tc = NPLANES - N_SC

    out_sc = _sc_hklut(img[n_tc:],
                       _sc_tables(h_msb, d_msb, b_msb, h_lsb, d_lsb))

    out = pl.pallas_call(
        _hklut_kernel,
        grid=(n_tc,),
        in_specs=[
            pl.BlockSpec((1, H, W), lambda p: (p, 0, 0)),
            pl.BlockSpec((4 * len(MSB_TERMS), 8, 128), lambda p: (0, 0, 0)),
            pl.BlockSpec((4 * len(LSB_TERMS), 8, 128), lambda p: (0, 0, 0)),
        ],
        out_specs=pl.BlockSpec((1, 2 * H, 2 * W), lambda p: (p, 0, 0)),
        out_shape=jax.ShapeDtypeStruct((n_tc, 2 * H, 2 * W), jnp.float32),
        scratch_shapes=[
            pltpu.VMEM((len(_CM_DYS), H, W + 4), jnp.int32),
            pltpu.VMEM((len(_CL_DYS), H, W + 4), jnp.int32),
        ],
        compiler_params=pltpu.CompilerParams(
            dimension_semantics=("arbitrary",),
        ),
    )(img[:n_tc], tm, tl)
    out = jnp.concatenate([out, out_sc], axis=0)
    return out.reshape(B, C, 2 * H, 2 * W)


# speedup vs baseline: 461.3432x; 1.0359x over previous
import dataclasses
import functools

import jax
import jax.numpy as jnp
from jax import lax
from jax.experimental import pallas as pl
from jax.experimental.pallas import tpu as pltpu
from jax.experimental.pallas import tpu_sc as plsc

H = W = 384
NPLANES = 24
N_SC = 20
_PERMS = ((0, 1, 2, 3), (2, 0, 3, 1), (3, 2, 1, 0), (1, 3, 0, 2))


def _delta(k, off):
    dy, dx = off
    return ((dy, dx), (dx, -dy), (-dy, -dx), (-dx, dy))[k]


_MSB_OFFS = ((0, 1), (1, 1), (1, 2))
_LSB_OFFS = ((0, 1), (1, 1))
MSB_TERMS = tuple(_delta(k, off) for off in _MSB_OFFS for k in range(4))
LSB_TERMS = tuple(_delta(k, off) for off in _LSB_OFFS for k in range(4))


def _pack_tables(luts, scale):
    rows = []
    for lut in luts:
        for k in range(4):
            eff = lut[:, list(_PERMS[k])] * scale
            b = eff.astype(jnp.bfloat16)
            u = lax.bitcast_convert_type(b, jnp.uint16).astype(jnp.uint32)
            p01 = (u[:, 0] << 16) | u[:, 1]
            p23 = (u[:, 2] << 16) | u[:, 3]
            rows += [p01[:128], p01[128:], p23[:128], p23[128:]]
    return jnp.stack(rows)


_CM_DYS = (-2, -1, 0, 1, 2)
_CL_DYS = (-1, 0, 1)

_DIR_MAP = {}
for _ti, _d in enumerate(MSB_TERMS):
    _DIR_MAP[_d] = [_ti, None]
for _li, _d in enumerate(LSB_TERMS):
    _DIR_MAP[_d][1] = _li
_DIRS = tuple((dy, dx, v[0], v[1]) for (dy, dx), v in _DIR_MAP.items())


def _sc_tables(h_msb, d_msb, b_msb, h_lsb, d_lsb):
    cols = []
    for luts, scale in (((h_msb, d_msb, b_msb), 1.0 / (12.0 * 255.0)),
                        ((h_lsb, d_lsb), 1.0 / (8.0 * 255.0))):
        for lut in luts:
            for k in range(4):
                cols.append((lut[:, list(_PERMS[k])] * scale).T)
    return jnp.concatenate(cols, axis=1)


def _sc_compiler_params():
    cp = pltpu.CompilerParams()
    if "needs_layout_passes" in pltpu.CompilerParams.__dataclass_fields__:
        cp = dataclasses.replace(cp, needs_layout_passes=False)
    return cp


def _sc_hklut(img_sc, tbl):
    n_sc = img_sc.shape[0]
    info = plsc.get_sparse_core_info()
    nc = info.num_cores
    mesh = plsc.VectorSubcoreMesh(core_axis_name="c", subcore_axis_name="s")

    @functools.partial(
        pl.kernel,
        out_type=jax.ShapeDtypeStruct((n_sc, 2 * H, 2 * W), jnp.float32),
        mesh=mesh,
        scratch_types=[
            pltpu.VMEM((24, W), jnp.float32),
            pltpu.VMEM((24, W + 16), jnp.int32),
            pltpu.VMEM((24, 2 * W), jnp.float32),
            pltpu.VMEM((4, 5120), jnp.float32),
        ],
        compiler_params=_sc_compiler_params(),
    )
    def k(img_hbm, tbl_hbm, out_hbm, imgc, xic, outc, tblv):
        wid = lax.axis_index("s") * nc + lax.axis_index("c")
        r0 = wid * 12
        base = pl.multiple_of(jnp.clip(((r0 - 2) // 8) * 8, 0, H - 24), 8)
        off = r0 - base
        iota = lax.iota(jnp.int32, 16)
        iota2 = iota * 2
        zero = jnp.zeros((16,), jnp.int32)
        iota_dx = {dx: iota + (dx + 2) for dx in _CM_DYS}
        cjs = (zero, zero + 1, zero + 2, zero + 3)
        pltpu.sync_copy(tbl_hbm, tblv)

        @pl.loop(0, n_sc)
        def _plane(p):
            pltpu.sync_copy(img_hbm.at[p, pl.ds(base, 24)], imgc)

            @pl.loop(0, 24)
            def _r(r):
                rv = zero + r

                @pl.loop(0, (W + 16) // 16)
                def _c(cc):
                    cv = cc * 16 + iota
                    src = jnp.clip(cv - 2, 0, W - 1)
                    xv = plsc.load_gather(imgc, [rv, src])
                    xiv = (xv * 255.0).astype(jnp.int32)
                    plsc.store_scatter(xic, [rv, cv], xiv)

            @pl.loop(0, 12)
            def _row(y):
                yy = y + off
                rowvs = {dy: jnp.clip(zero + (yy + dy), 0, 23)
                         for dy in _CM_DYS}
                rowe = zero + 2 * y
                rowo = rowe + 1

                @pl.loop(0, W // 16)
                def _x(xx):
                    xb = xx * 16
                    colvs = {dx: xb + iota_dx[dx] for dx in _CM_DYS}
                    rv0 = rowvs[0]
                    xv = plsc.load_gather(imgc, [rv0, xb + iota])
                    xiv = plsc.load_gather(xic, [rv0, colvs[0]])
                    cmb = (xiv >> 4) << 4
                    clb = (xiv & 15) << 4
                    accs = [xv, xv, xv, xv]
                    for (dy, dx, mt, lt) in _DIRS:
                        nxi = plsc.load_gather(xic, [rowvs[dy], colvs[dx]])
                        cols = []
                        if mt is not None:
                            cols.append(cmb + (nxi >> 4) + (mt * 256))
                        if lt is not None:
                            cols.append(clb + (nxi & 15) + ((12 + lt) * 256))
                        for col in cols:
                            for j in range(4):
                                g = plsc.load_gather(tblv, [cjs[j], col])
                                accs[j] = accs[j] + g
                    accs = [jnp.clip(a, 0.0, 1.0) for a in accs]
                    ce = xb * 2 + iota2
                    plsc.store_scatter(outc, [rowe, ce], accs[0])
                    plsc.store_scatter(outc, [rowe, ce + 1], accs[1])
                    plsc.store_scatter(outc, [rowo, ce], accs[2])
                    plsc.store_scatter(outc, [rowo, ce + 1], accs[3])

            pltpu.sync_copy(outc, out_hbm.at[p, pl.ds(pl.multiple_of(2 * r0, 8), 24)])

    return k(img_sc, tbl)


def _hklut_kernel(img_ref, tm_ref, tl_ref, out_ref, cmp_ref, clp_ref):
    x_plane = img_ref[0]
    xi = (x_plane * 255.0).astype(jnp.int32)
    cm = xi >> 4
    cl = xi & 15

    def padcols(a):
        return jnp.concatenate([a[:, :1], a[:, :1], a, a[:, -1:], a[:, -1:]],
                               axis=1)

    def rowshift(a, dy):
        if dy == 0:
            return a
        if dy > 0:
            return jnp.concatenate([a[dy:]] + dy * [a[-1:]], axis=0)
        return jnp.concatenate((-dy) * [a[:1]] + [a[:dy]], axis=0)

    for ref, plane, dys in ((cmp_ref, cm, _CM_DYS), (clp_ref, cl, _CL_DYS)):
        for j, dy in enumerate(dys):
            ref[j] = padcols(rowshift(plane, dy))

    def gather_l(tbl, idx):
        return jnp.take_along_axis(tbl, idx, axis=1, mode="promise_in_bounds")

    def gather_s(arr, idx):
        return jnp.take_along_axis(arr, idx, axis=0, mode="promise_in_bounds")

    lane = lax.broadcasted_iota(jnp.int32, (8, 128), 1)
    sub = lax.broadcasted_iota(jnp.int32, (8, 128), 0)
    lane_par = (lane & 1) == 1
    ilane = lane >> 1
    ilane_r = ilane + 64
    sub_par = (sub & 1) == 1
    isub_t = sub >> 1
    isub_b = isub_t + 4

    COLS = (0, 128, 256)

    def strip(i, carry):
        r0 = i * 8
        for c0 in COLS:
            x = img_ref[0, pl.ds(r0, 8), pl.ds(c0, 128)]
            xi_t = (x * 255.0).astype(jnp.int32)
            cm_c = xi_t >> 4
            cl_c = xi_t & 15
            mm = cm_c >= 8
            ml = cl_c >= 8
            cm716 = (cm_c & 7) << 4
            cl716 = (cl_c & 7) << 4
            acc = [x, x, x, x]

            for terms, tref, pad_ref, dys, base16, msk in (
                (MSB_TERMS, tm_ref, cmp_ref, _CM_DYS, cm716, mm),
                (LSB_TERMS, tl_ref, clp_ref, _CL_DYS, cl716, ml),
            ):
                for t, (dy, dx) in enumerate(terms):
                    nb = pad_ref[dys.index(dy), pl.ds(r0, 8),
                                 pl.ds(2 + c0 + dx, 128)]
                    idx7 = base16 | nb
                    for pair in (0, 1):
                        tlo = tref[4 * t + 2 * pair]
                        thi = tref[4 * t + 2 * pair + 1]
                        g = jnp.where(msk, gather_l(thi, idx7),
                                      gather_l(tlo, idx7))
                        acc[2 * pair] += lax.bitcast_convert_type(
                            g & jnp.uint32(0xFFFF0000), jnp.float32)
                        acc[2 * pair + 1] += lax.bitcast_convert_type(
                            g << 16, jnp.float32)

            acc = [jnp.clip(a, 0.0, 1.0) for a in acc]
            el = jnp.where(lane_par, gather_l(acc[1], ilane),
                           gather_l(acc[0], ilane))
            er = jnp.where(lane_par,
                           gather_l(pltpu.roll(acc[1], 64, 1), ilane),
                           gather_l(pltpu.roll(acc[0], 64, 1), ilane))
            ol = jnp.where(lane_par, gather_l(acc[3], ilane),
                           gather_l(acc[2], ilane))
            orr = jnp.where(lane_par,
                            gather_l(pltpu.roll(acc[3], 64, 1), ilane),
                            gather_l(pltpu.roll(acc[2], 64, 1), ilane))
            tlv = jnp.where(sub_par, gather_s(ol, isub_t), gather_s(el, isub_t))
            trv = jnp.where(sub_par, gather_s(orr, isub_t), gather_s(er, isub_t))
            blv = jnp.where(sub_par, gather_s(ol, isub_b), gather_s(el, isub_b))
            brv = jnp.where(sub_par, gather_s(orr, isub_b), gather_s(er, isub_b))
            out_ref[0, pl.ds(2 * r0, 8), pl.ds(2 * c0, 128)] = tlv
            out_ref[0, pl.ds(2 * r0, 8), pl.ds(2 * c0 + 128, 128)] = trv
            out_ref[0, pl.ds(2 * r0 + 8, 8), pl.ds(2 * c0, 128)] = blv
            out_ref[0, pl.ds(2 * r0 + 8, 8), pl.ds(2 * c0 + 128, 128)] = brv
        return carry

    lax.fori_loop(0, H // 8, strip, 0)


@jax.jit
def kernel(img_lr, h_msb, d_msb, b_msb, h_lsb, d_lsb):
    B, C = img_lr.shape[0], img_lr.shape[1]
    tm = _pack_tables((h_msb, d_msb, b_msb), 1.0 / (12.0 * 255.0))
    tl = _pack_tables((h_lsb, d_lsb), 1.0 / (8.0 * 255.0))
    tm = jnp.broadcast_to(tm[:, None, :], (tm.shape[0], 8, 128))
    tl = jnp.broadcast_to(tl[:, None, :], (tl.shape[0], 8, 128))
    img = img_lr.reshape(NPLANES, H, W)
    n_tc = NPLANES - N_SC

    out_sc = _sc_hklut(img[n_tc:],
                       _sc_tables(h_msb, d_msb, b_msb, h_lsb, d_lsb))

    out = pl.pallas_call(
        _hklut_kernel,
        grid=(n_tc,),
        in_specs=[
            pl.BlockSpec((1, H, W), lambda p: (p, 0, 0)),
            pl.BlockSpec((4 * len(MSB_TERMS), 8, 128), lambda p: (0, 0, 0)),
            pl.BlockSpec((4 * len(LSB_TERMS), 8, 128), lambda p: (0, 0, 0)),
        ],
        out_specs=pl.BlockSpec((1, 2 * H, 2 * W), lambda p: (p, 0, 0)),
        out_shape=jax.ShapeDtypeStruct((n_tc, 2 * H, 2 * W), jnp.float32),
        scratch_shapes=[
            pltpu.VMEM((len(_CM_DYS), H, W + 4), jnp.int32),
            pltpu.VMEM((len(_CL_DYS), H, W + 4), jnp.int32),
        ],
        compiler_params=pltpu.CompilerParams(
            dimension_semantics=("arbitrary",),
        ),
    )(img[:n_tc], tm, tl)
    out = jnp.concatenate([out, out_sc], axis=0)
    return out.reshape(B, C, 2 * H, 2 * W)


# speedup vs baseline: 485.9647x; 1.0534x over previous
import dataclasses
import functools

import jax
import jax.numpy as jnp
from jax import lax
from jax.experimental import pallas as pl
from jax.experimental.pallas import tpu as pltpu
from jax.experimental.pallas import tpu_sc as plsc

H = W = 384
NPLANES = 24
N_SC = 20
_PERMS = ((0, 1, 2, 3), (2, 0, 3, 1), (3, 2, 1, 0), (1, 3, 0, 2))


def _delta(k, off):
    dy, dx = off
    return ((dy, dx), (dx, -dy), (-dy, -dx), (-dx, dy))[k]


_MSB_OFFS = ((0, 1), (1, 1), (1, 2))
_LSB_OFFS = ((0, 1), (1, 1))
MSB_TERMS = tuple(_delta(k, off) for off in _MSB_OFFS for k in range(4))
LSB_TERMS = tuple(_delta(k, off) for off in _LSB_OFFS for k in range(4))


def _pack_tables(luts, scale):
    rows = []
    for lut in luts:
        for k in range(4):
            eff = lut[:, list(_PERMS[k])] * scale
            b = eff.astype(jnp.bfloat16)
            u = lax.bitcast_convert_type(b, jnp.uint16).astype(jnp.uint32)
            p01 = (u[:, 0] << 16) | u[:, 1]
            p23 = (u[:, 2] << 16) | u[:, 3]
            rows += [p01[:128], p01[128:], p23[:128], p23[128:]]
    return jnp.stack(rows)


_CM_DYS = (-2, -1, 0, 1, 2)
_CL_DYS = (-1, 0, 1)

_DIR_MAP = {}
for _ti, _d in enumerate(MSB_TERMS):
    _DIR_MAP[_d] = [_ti, None]
for _li, _d in enumerate(LSB_TERMS):
    _DIR_MAP[_d][1] = _li
_DIRS = tuple((dy, dx, v[0], v[1]) for (dy, dx), v in _DIR_MAP.items())


def _sc_tables(h_msb, d_msb, b_msb, h_lsb, d_lsb):
    cols = []
    for luts, scale in (((h_msb, d_msb, b_msb), 1.0 / (12.0 * 255.0)),
                        ((h_lsb, d_lsb), 1.0 / (8.0 * 255.0))):
        for lut in luts:
            for k in range(4):
                eff = lut[:, list(_PERMS[k])] * scale
                b = eff.astype(jnp.bfloat16)
                u = lax.bitcast_convert_type(b, jnp.uint16).astype(jnp.uint32)
                p01 = (u[:, 0] << 16) | u[:, 1]
                p23 = (u[:, 2] << 16) | u[:, 3]
                cols.append(jnp.stack([p01, p23]))
    return lax.bitcast_convert_type(jnp.concatenate(cols, axis=1), jnp.int32)


def _sc_compiler_params():
    cp = pltpu.CompilerParams()
    if "needs_layout_passes" in pltpu.CompilerParams.__dataclass_fields__:
        cp = dataclasses.replace(cp, needs_layout_passes=False)
    return cp


def _sc_hklut(img_sc, tbl):
    n_sc = img_sc.shape[0]
    info = plsc.get_sparse_core_info()
    nc = info.num_cores
    mesh = plsc.VectorSubcoreMesh(core_axis_name="c", subcore_axis_name="s")

    @functools.partial(
        pl.kernel,
        out_type=jax.ShapeDtypeStruct((n_sc, 2 * H, 2 * W), jnp.float32),
        mesh=mesh,
        scratch_types=[
            pltpu.VMEM((24, W), jnp.float32),
            pltpu.VMEM((24, W + 16), jnp.int32),
            pltpu.VMEM((24, 2 * W), jnp.float32),
            pltpu.VMEM((2, 5120), jnp.int32),
        ],
        compiler_params=_sc_compiler_params(),
    )
    def k(img_hbm, tbl_hbm, out_hbm, imgc, xic, outc, tblv):
        wid = lax.axis_index("s") * nc + lax.axis_index("c")
        r0 = wid * 12
        base = pl.multiple_of(jnp.clip(((r0 - 2) // 8) * 8, 0, H - 24), 8)
        off = r0 - base
        iota = lax.iota(jnp.int32, 16)
        iota2 = iota * 2
        zero = jnp.zeros((16,), jnp.int32)
        iota_dx = {dx: iota + (dx + 2) for dx in _CM_DYS}
        cjs = (zero, zero + 1)
        himask = jnp.full((16,), -65536, jnp.int32)
        pltpu.sync_copy(tbl_hbm, tblv)

        @pl.loop(0, n_sc)
        def _plane(p):
            pltpu.sync_copy(img_hbm.at[p, pl.ds(base, 24)], imgc)

            @pl.loop(0, 24)
            def _r(r):
                rv = zero + r

                @pl.loop(0, (W + 16) // 16)
                def _c(cc):
                    cv = cc * 16 + iota
                    src = jnp.clip(cv - 2, 0, W - 1)
                    xv = plsc.load_gather(imgc, [rv, src])
                    xiv = (xv * 255.0).astype(jnp.int32)
                    plsc.store_scatter(xic, [rv, cv], xiv)

            @pl.loop(0, 12)
            def _row(y):
                yy = y + off
                rowvs = {dy: jnp.clip(zero + (yy + dy), 0, 23)
                         for dy in _CM_DYS}
                rowe = zero + 2 * y
                rowo = rowe + 1

                @pl.loop(0, W // 16)
                def _x(xx):
                    xb = xx * 16
                    colvs = {dx: xb + iota_dx[dx] for dx in _CM_DYS}
                    rv0 = rowvs[0]
                    xv = plsc.load_gather(imgc, [rv0, xb + iota])
                    xiv = plsc.load_gather(xic, [rv0, colvs[0]])
                    cmb = (xiv >> 4) << 4
                    clb = (xiv & 15) << 4
                    accs = [xv, xv, xv, xv]
                    for (dy, dx, mt, lt) in _DIRS:
                        nxi = plsc.load_gather(xic, [rowvs[dy], colvs[dx]])
                        cols = []
                        if mt is not None:
                            cols.append(cmb + (nxi >> 4) + (mt * 256))
                        if lt is not None:
                            cols.append(clb + (nxi & 15) + ((12 + lt) * 256))
                        for col in cols:
                            for pr in (0, 1):
                                g = plsc.load_gather(tblv, [cjs[pr], col])
                                accs[2 * pr] = accs[2 * pr] + plsc.bitcast(
                                    g & himask, jnp.float32)
                                accs[2 * pr + 1] = accs[2 * pr + 1] + plsc.bitcast(
                                    g << 16, jnp.float32)
                    accs = [jnp.clip(a, 0.0, 1.0) for a in accs]
                    ce = xb * 2 + iota2
                    plsc.store_scatter(outc, [rowe, ce], accs[0])
                    plsc.store_scatter(outc, [rowe, ce + 1], accs[1])
                    plsc.store_scatter(outc, [rowo, ce], accs[2])
                    plsc.store_scatter(outc, [rowo, ce + 1], accs[3])

            pltpu.sync_copy(outc, out_hbm.at[p, pl.ds(pl.multiple_of(2 * r0, 8), 24)])

    return k(img_sc, tbl)


def _hklut_kernel(img_ref, tm_ref, tl_ref, out_ref, cmp_ref, clp_ref):
    x_plane = img_ref[0]
    xi = (x_plane * 255.0).astype(jnp.int32)
    cm = xi >> 4
    cl = xi & 15

    def padcols(a):
        return jnp.concatenate([a[:, :1], a[:, :1], a, a[:, -1:], a[:, -1:]],
                               axis=1)

    def rowshift(a, dy):
        if dy == 0:
            return a
        if dy > 0:
            return jnp.concatenate([a[dy:]] + dy * [a[-1:]], axis=0)
        return jnp.concatenate((-dy) * [a[:1]] + [a[:dy]], axis=0)

    for ref, plane, dys in ((cmp_ref, cm, _CM_DYS), (clp_ref, cl, _CL_DYS)):
        for j, dy in enumerate(dys):
            ref[j] = padcols(rowshift(plane, dy))

    def gather_l(tbl, idx):
        return jnp.take_along_axis(tbl, idx, axis=1, mode="promise_in_bounds")

    def gather_s(arr, idx):
        return jnp.take_along_axis(arr, idx, axis=0, mode="promise_in_bounds")

    lane = lax.broadcasted_iota(jnp.int32, (8, 128), 1)
    sub = lax.broadcasted_iota(jnp.int32, (8, 128), 0)
    lane_par = (lane & 1) == 1
    ilane = lane >> 1
    ilane_r = ilane + 64
    sub_par = (sub & 1) == 1
    isub_t = sub >> 1
    isub_b = isub_t + 4

    COLS = (0, 128, 256)

    def strip(i, carry):
        r0 = i * 8
        for c0 in COLS:
            x = img_ref[0, pl.ds(r0, 8), pl.ds(c0, 128)]
            xi_t = (x * 255.0).astype(jnp.int32)
            cm_c = xi_t >> 4
            cl_c = xi_t & 15
            mm = cm_c >= 8
            ml = cl_c >= 8
            cm716 = (cm_c & 7) << 4
            cl716 = (cl_c & 7) << 4
            acc = [x, x, x, x]

            for terms, tref, pad_ref, dys, base16, msk in (
                (MSB_TERMS, tm_ref, cmp_ref, _CM_DYS, cm716, mm),
                (LSB_TERMS, tl_ref, clp_ref, _CL_DYS, cl716, ml),
            ):
                for t, (dy, dx) in enumerate(terms):
                    nb = pad_ref[dys.index(dy), pl.ds(r0, 8),
                                 pl.ds(2 + c0 + dx, 128)]
                    idx7 = base16 | nb
                    for pair in (0, 1):
                        tlo = tref[4 * t + 2 * pair]
                        thi = tref[4 * t + 2 * pair + 1]
                        g = jnp.where(msk, gather_l(thi, idx7),
                                      gather_l(tlo, idx7))
                        acc[2 * pair] += lax.bitcast_convert_type(
                            g & jnp.uint32(0xFFFF0000), jnp.float32)
                        acc[2 * pair + 1] += lax.bitcast_convert_type(
                            g << 16, jnp.float32)

            acc = [jnp.clip(a, 0.0, 1.0) for a in acc]
            el = jnp.where(lane_par, gather_l(acc[1], ilane),
                           gather_l(acc[0], ilane))
            er = jnp.where(lane_par,
                           gather_l(pltpu.roll(acc[1], 64, 1), ilane),
                           gather_l(pltpu.roll(acc[0], 64, 1), ilane))
            ol = jnp.where(lane_par, gather_l(acc[3], ilane),
                           gather_l(acc[2], ilane))
            orr = jnp.where(lane_par,
                            gather_l(pltpu.roll(acc[3], 64, 1), ilane),
                            gather_l(pltpu.roll(acc[2], 64, 1), ilane))
            tlv = jnp.where(sub_par, gather_s(ol, isub_t), gather_s(el, isub_t))
            trv = jnp.where(sub_par, gather_s(orr, isub_t), gather_s(er, isub_t))
            blv = jnp.where(sub_par, gather_s(ol, isub_b), gather_s(el, isub_b))
            brv = jnp.where(sub_par, gather_s(orr, isub_b), gather_s(er, isub_b))
            out_ref[0, pl.ds(2 * r0, 8), pl.ds(2 * c0, 128)] = tlv
            out_ref[0, pl.ds(2 * r0, 8), pl.ds(2 * c0 + 128, 128)] = trv
            out_ref[0, pl.ds(2 * r0 + 8, 8), pl.ds(2 * c0, 128)] = blv
            out_ref[0, pl.ds(2 * r0 + 8, 8), pl.ds(2 * c0 + 128, 128)] = brv
        return carry

    lax.fori_loop(0, H // 8, strip, 0)


@jax.jit
def kernel(img_lr, h_msb, d_msb, b_msb, h_lsb, d_lsb):
    B, C = img_lr.shape[0], img_lr.shape[1]
    tm = _pack_tables((h_msb, d_msb, b_msb), 1.0 / (12.0 * 255.0))
    tl = _pack_tables((h_lsb, d_lsb), 1.0 / (8.0 * 255.0))
    tm = jnp.broadcast_to(tm[:, None, :], (tm.shape[0], 8, 128))
    tl = jnp.broadcast_to(tl[:, None, :], (tl.shape[0], 8, 128))
    img = img_lr.reshape(NPLANES, H, W)
    n_tc = NPLANES - N_SC

    out_sc = _sc_hklut(img[n_tc:],
                       _sc_tables(h_msb, d_msb, b_msb, h_lsb, d_lsb))

    out = pl.pallas_call(
        _hklut_kernel,
        grid=(n_tc,),
        in_specs=[
            pl.BlockSpec((1, H, W), lambda p: (p, 0, 0)),
            pl.BlockSpec((4 * len(MSB_TERMS), 8, 128), lambda p: (0, 0, 0)),
            pl.BlockSpec((4 * len(LSB_TERMS), 8, 128), lambda p: (0, 0, 0)),
        ],
        out_specs=pl.BlockSpec((1, 2 * H, 2 * W), lambda p: (p, 0, 0)),
        out_shape=jax.ShapeDtypeStruct((n_tc, 2 * H, 2 * W), jnp.float32),
        scratch_shapes=[
            pltpu.VMEM((len(_CM_DYS), H, W + 4), jnp.int32),
            pltpu.VMEM((len(_CL_DYS), H, W + 4), jnp.int32),
        ],
        compiler_params=pltpu.CompilerParams(
            dimension_semantics=("arbitrary",),
        ),
    )(img[:n_tc], tm, tl)
    out = jnp.concatenate([out, out_sc], axis=0)
    return out.reshape(B, C, 2 * H, 2 * W)


# speedup vs baseline: 489.2728x; 1.0068x over previous
import dataclasses
import functools

import jax
import jax.numpy as jnp
from jax import lax
from jax.experimental import pallas as pl
from jax.experimental.pallas import tpu as pltpu
from jax.experimental.pallas import tpu_sc as plsc

H = W = 384
NPLANES = 24
N_SC = 20
_PERMS = ((0, 1, 2, 3), (2, 0, 3, 1), (3, 2, 1, 0), (1, 3, 0, 2))


def _delta(k, off):
    dy, dx = off
    return ((dy, dx), (dx, -dy), (-dy, -dx), (-dx, dy))[k]


_MSB_OFFS = ((0, 1), (1, 1), (1, 2))
_LSB_OFFS = ((0, 1), (1, 1))
MSB_TERMS = tuple(_delta(k, off) for off in _MSB_OFFS for k in range(4))
LSB_TERMS = tuple(_delta(k, off) for off in _LSB_OFFS for k in range(4))


def _pack_tables(luts, scale):
    rows = []
    for lut in luts:
        for k in range(4):
            eff = lut[:, list(_PERMS[k])] * scale
            b = eff.astype(jnp.bfloat16)
            u = lax.bitcast_convert_type(b, jnp.uint16).astype(jnp.uint32)
            p01 = (u[:, 0] << 16) | u[:, 1]
            p23 = (u[:, 2] << 16) | u[:, 3]
            rows += [p01[:128], p01[128:], p23[:128], p23[128:]]
    return jnp.stack(rows)


_CM_DYS = (-2, -1, 0, 1, 2)
_CL_DYS = (-1, 0, 1)

_DIR_MAP = {}
for _ti, _d in enumerate(MSB_TERMS):
    _DIR_MAP[_d] = [_ti, None]
for _li, _d in enumerate(LSB_TERMS):
    _DIR_MAP[_d][1] = _li
_DIRS = tuple((dy, dx, v[0], v[1]) for (dy, dx), v in _DIR_MAP.items())


def _sc_tables(h_msb, d_msb, b_msb, h_lsb, d_lsb):
    cols = []
    for luts, scale in (((h_msb, d_msb, b_msb), 1.0 / (12.0 * 255.0)),
                        ((h_lsb, d_lsb), 1.0 / (8.0 * 255.0))):
        for lut in luts:
            for k in range(4):
                eff = lut[:, list(_PERMS[k])] * scale
                b = eff.astype(jnp.bfloat16)
                u = lax.bitcast_convert_type(b, jnp.uint16).astype(jnp.uint32)
                p01 = (u[:, 0] << 16) | u[:, 1]
                p23 = (u[:, 2] << 16) | u[:, 3]
                cols.append(jnp.stack([p01, p23]))
    return lax.bitcast_convert_type(jnp.concatenate(cols, axis=1), jnp.int32)


def _sc_compiler_params():
    cp = pltpu.CompilerParams()
    if "needs_layout_passes" in pltpu.CompilerParams.__dataclass_fields__:
        cp = dataclasses.replace(cp, needs_layout_passes=False)
    return cp


def _sc_hklut(img_sc, tbl):
    n_sc = img_sc.shape[0]
    info = plsc.get_sparse_core_info()
    nc = info.num_cores
    mesh = plsc.VectorSubcoreMesh(core_axis_name="c", subcore_axis_name="s")

    @functools.partial(
        pl.kernel,
        out_type=jax.ShapeDtypeStruct((n_sc, 2 * H, 2 * W), jnp.float32),
        mesh=mesh,
        scratch_types=[
            pltpu.VMEM((24, W), jnp.float32),
            pltpu.VMEM((24, W + 16), jnp.int32),
            pltpu.VMEM((24, 2 * W), jnp.float32),
            pltpu.VMEM((2, 5120), jnp.int32),
        ],
        compiler_params=_sc_compiler_params(),
    )
    def k(img_hbm, tbl_hbm, out_hbm, imgc, xic, outc, tblv):
        wid = lax.axis_index("s") * nc + lax.axis_index("c")
        r0 = wid * 12
        base = pl.multiple_of(jnp.clip(((r0 - 2) // 8) * 8, 0, H - 24), 8)
        off = r0 - base
        iota = lax.iota(jnp.int32, 16)
        iota2 = iota * 2
        zero = jnp.zeros((16,), jnp.int32)
        iota_dx = {dx: iota + (dx + 2) for dx in _CM_DYS}
        cjs = (zero, zero + 1)
        himask = jnp.full((16,), -65536, jnp.int32)
        pltpu.sync_copy(tbl_hbm, tblv)

        @pl.loop(0, n_sc)
        def _plane(p):
            pltpu.sync_copy(img_hbm.at[p, pl.ds(base, 24)], imgc)

            @pl.loop(0, 24)
            def _r(r):
                rv = zero + r

                @pl.loop(0, (W + 16) // 16)
                def _c(cc):
                    cv = cc * 16 + iota
                    src = jnp.clip(cv - 2, 0, W - 1)
                    xv = plsc.load_gather(imgc, [rv, src])
                    xiv = (xv * 255.0).astype(jnp.int32)
                    plsc.store_scatter(xic, [rv, cv], xiv)

            @pl.loop(0, 12)
            def _row(y):
                yy = y + off
                rowvs = {dy: jnp.clip(zero + (yy + dy), 0, 23)
                         for dy in _CM_DYS}
                rowe = zero + 2 * y
                rowo = rowe + 1

                @plsc.parallel_loop(0, W // 16, unroll=2)
                def _x(xx):
                    xb = xx * 16
                    colvs = {dx: xb + iota_dx[dx] for dx in _CM_DYS}
                    rv0 = rowvs[0]
                    xv = plsc.load_gather(imgc, [rv0, xb + iota])
                    xiv = plsc.load_gather(xic, [rv0, colvs[0]])
                    cmb = (xiv >> 4) << 4
                    clb = (xiv & 15) << 4
                    accs = [xv, xv, xv, xv]
                    for (dy, dx, mt, lt) in _DIRS:
                        nxi = plsc.load_gather(xic, [rowvs[dy], colvs[dx]])
                        cols = []
                        if mt is not None:
                            cols.append(cmb + (nxi >> 4) + (mt * 256))
                        if lt is not None:
                            cols.append(clb + (nxi & 15) + ((12 + lt) * 256))
                        for col in cols:
                            for pr in (0, 1):
                                g = plsc.load_gather(tblv, [cjs[pr], col])
                                accs[2 * pr] = accs[2 * pr] + plsc.bitcast(
                                    g & himask, jnp.float32)
                                accs[2 * pr + 1] = accs[2 * pr + 1] + plsc.bitcast(
                                    g << 16, jnp.float32)
                    accs = [jnp.clip(a, 0.0, 1.0) for a in accs]
                    ce = xb * 2 + iota2
                    plsc.store_scatter(outc, [rowe, ce], accs[0])
                    plsc.store_scatter(outc, [rowe, ce + 1], accs[1])
                    plsc.store_scatter(outc, [rowo, ce], accs[2])
                    plsc.store_scatter(outc, [rowo, ce + 1], accs[3])

            pltpu.sync_copy(outc, out_hbm.at[p, pl.ds(pl.multiple_of(2 * r0, 8), 24)])

    return k(img_sc, tbl)


def _hklut_kernel(img_ref, tm_ref, tl_ref, out_ref, cmp_ref, clp_ref):
    x_plane = img_ref[0]
    xi = (x_plane * 255.0).astype(jnp.int32)
    cm = xi >> 4
    cl = xi & 15

    def padcols(a):
        return jnp.concatenate([a[:, :1], a[:, :1], a, a[:, -1:], a[:, -1:]],
                               axis=1)

    def rowshift(a, dy):
        if dy == 0:
            return a
        if dy > 0:
            return jnp.concatenate([a[dy:]] + dy * [a[-1:]], axis=0)
        return jnp.concatenate((-dy) * [a[:1]] + [a[:dy]], axis=0)

    for ref, plane, dys in ((cmp_ref, cm, _CM_DYS), (clp_ref, cl, _CL_DYS)):
        for j, dy in enumerate(dys):
            ref[j] = padcols(rowshift(plane, dy))

    def gather_l(tbl, idx):
        return jnp.take_along_axis(tbl, idx, axis=1, mode="promise_in_bounds")

    def gather_s(arr, idx):
        return jnp.take_along_axis(arr, idx, axis=0, mode="promise_in_bounds")

    lane = lax.broadcasted_iota(jnp.int32, (8, 128), 1)
    sub = lax.broadcasted_iota(jnp.int32, (8, 128), 0)
    lane_par = (lane & 1) == 1
    ilane = lane >> 1
    ilane_r = ilane + 64
    sub_par = (sub & 1) == 1
    isub_t = sub >> 1
    isub_b = isub_t + 4

    COLS = (0, 128, 256)

    def strip(i, carry):
        r0 = i * 8
        for c0 in COLS:
            x = img_ref[0, pl.ds(r0, 8), pl.ds(c0, 128)]
            xi_t = (x * 255.0).astype(jnp.int32)
            cm_c = xi_t >> 4
            cl_c = xi_t & 15
            mm = cm_c >= 8
            ml = cl_c >= 8
            cm716 = (cm_c & 7) << 4
            cl716 = (cl_c & 7) << 4
            acc = [x, x, x, x]

            for terms, tref, pad_ref, dys, base16, msk in (
                (MSB_TERMS, tm_ref, cmp_ref, _CM_DYS, cm716, mm),
                (LSB_TERMS, tl_ref, clp_ref, _CL_DYS, cl716, ml),
            ):
                for t, (dy, dx) in enumerate(terms):
                    nb = pad_ref[dys.index(dy), pl.ds(r0, 8),
                                 pl.ds(2 + c0 + dx, 128)]
                    idx7 = base16 | nb
                    for pair in (0, 1):
                        tlo = tref[4 * t + 2 * pair]
                        thi = tref[4 * t + 2 * pair + 1]
                        g = jnp.where(msk, gather_l(thi, idx7),
                                      gather_l(tlo, idx7))
                        acc[2 * pair] += lax.bitcast_convert_type(
                            g & jnp.uint32(0xFFFF0000), jnp.float32)
                        acc[2 * pair + 1] += lax.bitcast_convert_type(
                            g << 16, jnp.float32)

            acc = [jnp.clip(a, 0.0, 1.0) for a in acc]
            el = jnp.where(lane_par, gather_l(acc[1], ilane),
                           gather_l(acc[0], ilane))
            er = jnp.where(lane_par,
                           gather_l(pltpu.roll(acc[1], 64, 1), ilane),
                           gather_l(pltpu.roll(acc[0], 64, 1), ilane))
            ol = jnp.where(lane_par, gather_l(acc[3], ilane),
                           gather_l(acc[2], ilane))
            orr = jnp.where(lane_par,
                            gather_l(pltpu.roll(acc[3], 64, 1), ilane),
                            gather_l(pltpu.roll(acc[2], 64, 1), ilane))
            tlv = jnp.where(sub_par, gather_s(ol, isub_t), gather_s(el, isub_t))
            trv = jnp.where(sub_par, gather_s(orr, isub_t), gather_s(er, isub_t))
            blv = jnp.where(sub_par, gather_s(ol, isub_b), gather_s(el, isub_b))
            brv = jnp.where(sub_par, gather_s(orr, isub_b), gather_s(er, isub_b))
            out_ref[0, pl.ds(2 * r0, 8), pl.ds(2 * c0, 128)] = tlv
            out_ref[0, pl.ds(2 * r0, 8), pl.ds(2 * c0 + 128, 128)] = trv
            out_ref[0, pl.ds(2 * r0 + 8, 8), pl.ds(2 * c0, 128)] = blv
            out_ref[0, pl.ds(2 * r0 + 8, 8), pl.ds(2 * c0 + 128, 128)] = brv
        return carry

    lax.fori_loop(0, H // 8, strip, 0)


@jax.jit
def kernel(img_lr, h_msb, d_msb, b_msb, h_lsb, d_lsb):
    B, C = img_lr.shape[0], img_lr.shape[1]
    tm = _pack_tables((h_msb, d_msb, b_msb), 1.0 / (12.0 * 255.0))
    tl = _pack_tables((h_lsb, d_lsb), 1.0 / (8.0 * 255.0))
    tm = jnp.broadcast_to(tm[:, None, :], (tm.shape[0], 8, 128))
    tl = jnp.broadcast_to(tl[:, None, :], (tl.shape[0], 8, 128))
    img = img_lr.reshape(NPLANES, H, W)
    n_tc = NPLANES - N_SC

    out_sc = _sc_hklut(img[n_tc:],
                       _sc_tables(h_msb, d_msb, b_msb, h_lsb, d_lsb))

    out = pl.pallas_call(
        _hklut_kernel,
        grid=(n_tc,),
        in_specs=[
            pl.BlockSpec((1, H, W), lambda p: (p, 0, 0)),
            pl.BlockSpec((4 * len(MSB_TERMS), 8, 128), lambda p: (0, 0, 0)),
            pl.BlockSpec((4 * len(LSB_TERMS), 8, 128), lambda p: (0, 0, 0)),
        ],
        out_specs=pl.BlockSpec((1, 2 * H, 2 * W), lambda p: (p, 0, 0)),
        out_shape=jax.ShapeDtypeStruct((n_tc, 2 * H, 2 * W), jnp.float32),
        scratch_shapes=[
            pltpu.VMEM((len(_CM_DYS), H, W + 4), jnp.int32),
            pltpu.VMEM((len(_CL_DYS), H, W + 4), jnp.int32),
        ],
        compiler_params=pltpu.CompilerParams(
            dimension_semantics=("arbitrary",),
        ),
    )(img[:n_tc], tm, tl)
    out = jnp.concatenate([out, out_sc], axis=0)
    return out.reshape(B, C, 2 * H, 2 * W)


# speedup vs baseline: 510.5013x; 1.0434x over previous
import dataclasses
import functools

import jax
import jax.numpy as jnp
from jax import lax
from jax.experimental import pallas as pl
from jax.experimental.pallas import tpu as pltpu
from jax.experimental.pallas import tpu_sc as plsc

H = W = 384
NPLANES = 24
N_SC = 20
_PERMS = ((0, 1, 2, 3), (2, 0, 3, 1), (3, 2, 1, 0), (1, 3, 0, 2))


def _delta(k, off):
    dy, dx = off
    return ((dy, dx), (dx, -dy), (-dy, -dx), (-dx, dy))[k]


_MSB_OFFS = ((0, 1), (1, 1), (1, 2))
_LSB_OFFS = ((0, 1), (1, 1))
MSB_TERMS = tuple(_delta(k, off) for off in _MSB_OFFS for k in range(4))
LSB_TERMS = tuple(_delta(k, off) for off in _LSB_OFFS for k in range(4))


def _pack_tables(luts, scale):
    rows = []
    for lut in luts:
        for k in range(4):
            eff = lut[:, list(_PERMS[k])] * scale
            b = eff.astype(jnp.bfloat16)
            u = lax.bitcast_convert_type(b, jnp.uint16).astype(jnp.uint32)
            p01 = (u[:, 0] << 16) | u[:, 1]
            p23 = (u[:, 2] << 16) | u[:, 3]
            rows += [p01[:128], p01[128:], p23[:128], p23[128:]]
    return jnp.stack(rows)


_CM_DYS = (-2, -1, 0, 1, 2)
_CL_DYS = (-1, 0, 1)

_DIR_MAP = {}
for _ti, _d in enumerate(MSB_TERMS):
    _DIR_MAP[_d] = [_ti, None]
for _li, _d in enumerate(LSB_TERMS):
    _DIR_MAP[_d][1] = _li
_DIRS = tuple((dy, dx, v[0], v[1]) for (dy, dx), v in _DIR_MAP.items())


def _sc_tables(h_msb, d_msb, b_msb, h_lsb, d_lsb):
    cols = []
    for luts, scale in (((h_msb, d_msb, b_msb), 1.0 / (12.0 * 255.0)),
                        ((h_lsb, d_lsb), 1.0 / (8.0 * 255.0))):
        for lut in luts:
            for k in range(4):
                eff = lut[:, list(_PERMS[k])] * scale
                b = eff.astype(jnp.bfloat16)
                u = lax.bitcast_convert_type(b, jnp.uint16).astype(jnp.uint32)
                p01 = (u[:, 0] << 16) | u[:, 1]
                p23 = (u[:, 2] << 16) | u[:, 3]
                cols.append(jnp.stack([p01, p23]))
    return lax.bitcast_convert_type(jnp.concatenate(cols, axis=1), jnp.int32)


def _sc_compiler_params():
    cp = pltpu.CompilerParams()
    if "needs_layout_passes" in pltpu.CompilerParams.__dataclass_fields__:
        cp = dataclasses.replace(cp, needs_layout_passes=False)
    return cp


def _sc_hklut(img_sc, tbl):
    n_sc = img_sc.shape[0]
    info = plsc.get_sparse_core_info()
    nc = info.num_cores
    mesh = plsc.VectorSubcoreMesh(core_axis_name="c", subcore_axis_name="s")

    @functools.partial(
        pl.kernel,
        out_type=jax.ShapeDtypeStruct((n_sc, 2 * H, 2 * W), jnp.float32),
        mesh=mesh,
        scratch_types=[
            pltpu.VMEM((2, 24, W), jnp.float32),
            pltpu.VMEM((24, W + 16), jnp.int32),
            pltpu.VMEM((2, 24, 2 * W), jnp.float32),
            pltpu.VMEM((2, 5120), jnp.int32),
            pltpu.SemaphoreType.DMA,
            pltpu.SemaphoreType.DMA,
        ],
        compiler_params=_sc_compiler_params(),
    )
    def k(img_hbm, tbl_hbm, out_hbm, imgc, xic, outc, tblv, sem_in, sem_out):
        wid = lax.axis_index("s") * nc + lax.axis_index("c")
        r0 = wid * 12
        base = pl.multiple_of(jnp.clip(((r0 - 2) // 8) * 8, 0, H - 24), 8)
        off = r0 - base
        iota = lax.iota(jnp.int32, 16)
        iota2 = iota * 2
        zero = jnp.zeros((16,), jnp.int32)
        iota_dx = {dx: iota + (dx + 2) for dx in _CM_DYS}
        cjs = (zero, zero + 1)
        himask = jnp.full((16,), -65536, jnp.int32)
        pltpu.sync_copy(tbl_hbm, tblv)
        hr0 = pl.multiple_of(2 * r0, 8)
        pltpu.async_copy(img_hbm.at[0, pl.ds(base, 24)], imgc.at[0], sem_in)

        @pl.loop(0, n_sc)
        def _plane(p):
            b = p & 1
            pltpu.make_async_copy(img_hbm.at[p, pl.ds(base, 24)],
                                  imgc.at[b], sem_in).wait()

            @pl.when(p + 1 < n_sc)
            def _prefetch():
                pltpu.async_copy(img_hbm.at[p + 1, pl.ds(base, 24)],
                                 imgc.at[1 - b], sem_in)

            @pl.when(p >= 2)
            def _drain_out():
                pltpu.make_async_copy(outc.at[b],
                                      out_hbm.at[p - 2, pl.ds(hr0, 24)],
                                      sem_out).wait()

            @pl.loop(0, 24)
            def _r(r):
                rv = zero + r

                @pl.loop(0, (W + 16) // 16)
                def _c(cc):
                    cv = cc * 16 + iota
                    src = jnp.clip(cv - 2, 0, W - 1)
                    xv = plsc.load_gather(imgc.at[b], [rv, src])
                    xiv = (xv * 255.0).astype(jnp.int32)
                    plsc.store_scatter(xic, [rv, cv], xiv)

            @pl.loop(0, 12)
            def _row(y):
                yy = y + off
                rowvs = {dy: jnp.clip(zero + (yy + dy), 0, 23)
                         for dy in _CM_DYS}
                rowe = zero + 2 * y
                rowo = rowe + 1

                @plsc.parallel_loop(0, W // 16, unroll=2)
                def _x(xx):
                    xb = xx * 16
                    colvs = {dx: xb + iota_dx[dx] for dx in _CM_DYS}
                    rv0 = rowvs[0]
                    xv = plsc.load_gather(imgc.at[b], [rv0, xb + iota])
                    xiv = plsc.load_gather(xic, [rv0, colvs[0]])
                    cmb = (xiv >> 4) << 4
                    clb = (xiv & 15) << 4
                    accs = [xv, xv, xv, xv]
                    for (dy, dx, mt, lt) in _DIRS:
                        nxi = plsc.load_gather(xic, [rowvs[dy], colvs[dx]])
                        cols = []
                        if mt is not None:
                            cols.append(cmb + (nxi >> 4) + (mt * 256))
                        if lt is not None:
                            cols.append(clb + (nxi & 15) + ((12 + lt) * 256))
                        for col in cols:
                            for pr in (0, 1):
                                g = plsc.load_gather(tblv, [cjs[pr], col])
                                accs[2 * pr] = accs[2 * pr] + plsc.bitcast(
                                    g & himask, jnp.float32)
                                accs[2 * pr + 1] = accs[2 * pr + 1] + plsc.bitcast(
                                    g << 16, jnp.float32)
                    accs = [jnp.clip(a, 0.0, 1.0) for a in accs]
                    ce = xb * 2 + iota2
                    plsc.store_scatter(outc.at[b], [rowe, ce], accs[0])
                    plsc.store_scatter(outc.at[b], [rowe, ce + 1], accs[1])
                    plsc.store_scatter(outc.at[b], [rowo, ce], accs[2])
                    plsc.store_scatter(outc.at[b], [rowo, ce + 1], accs[3])

            pltpu.async_copy(outc.at[b], out_hbm.at[p, pl.ds(hr0, 24)],
                             sem_out)

        for i in range(2):
            pltpu.make_async_copy(outc.at[i], out_hbm.at[0, pl.ds(hr0, 24)],
                                  sem_out).wait()

    return k(img_sc, tbl)


def _hklut_kernel(img_ref, tm_ref, tl_ref, out_ref, cmp_ref, clp_ref):
    x_plane = img_ref[0]
    xi = (x_plane * 255.0).astype(jnp.int32)
    cm = xi >> 4
    cl = xi & 15

    def padcols(a):
        return jnp.concatenate([a[:, :1], a[:, :1], a, a[:, -1:], a[:, -1:]],
                               axis=1)

    def rowshift(a, dy):
        if dy == 0:
            return a
        if dy > 0:
            return jnp.concatenate([a[dy:]] + dy * [a[-1:]], axis=0)
        return jnp.concatenate((-dy) * [a[:1]] + [a[:dy]], axis=0)

    for ref, plane, dys in ((cmp_ref, cm, _CM_DYS), (clp_ref, cl, _CL_DYS)):
        for j, dy in enumerate(dys):
            ref[j] = padcols(rowshift(plane, dy))

    def gather_l(tbl, idx):
        return jnp.take_along_axis(tbl, idx, axis=1, mode="promise_in_bounds")

    def gather_s(arr, idx):
        return jnp.take_along_axis(arr, idx, axis=0, mode="promise_in_bounds")

    lane = lax.broadcasted_iota(jnp.int32, (8, 128), 1)
    sub = lax.broadcasted_iota(jnp.int32, (8, 128), 0)
    lane_par = (lane & 1) == 1
    ilane = lane >> 1
    ilane_r = ilane + 64
    sub_par = (sub & 1) == 1
    isub_t = sub >> 1
    isub_b = isub_t + 4

    COLS = (0, 128, 256)

    def strip(i, carry):
        r0 = i * 8
        for c0 in COLS:
            x = img_ref[0, pl.ds(r0, 8), pl.ds(c0, 128)]
            xi_t = (x * 255.0).astype(jnp.int32)
            cm_c = xi_t >> 4
            cl_c = xi_t & 15
            mm = cm_c >= 8
            ml = cl_c >= 8
            cm716 = (cm_c & 7) << 4
            cl716 = (cl_c & 7) << 4
            acc = [x, x, x, x]

            for terms, tref, pad_ref, dys, base16, msk in (
                (MSB_TERMS, tm_ref, cmp_ref, _CM_DYS, cm716, mm),
                (LSB_TERMS, tl_ref, clp_ref, _CL_DYS, cl716, ml),
            ):
                for t, (dy, dx) in enumerate(terms):
                    nb = pad_ref[dys.index(dy), pl.ds(r0, 8),
                                 pl.ds(2 + c0 + dx, 128)]
                    idx7 = base16 | nb
                    for pair in (0, 1):
                        tlo = tref[4 * t + 2 * pair]
                        thi = tref[4 * t + 2 * pair + 1]
                        g = jnp.where(msk, gather_l(thi, idx7),
                                      gather_l(tlo, idx7))
                        acc[2 * pair] += lax.bitcast_convert_type(
                            g & jnp.uint32(0xFFFF0000), jnp.float32)
                        acc[2 * pair + 1] += lax.bitcast_convert_type(
                            g << 16, jnp.float32)

            acc = [jnp.clip(a, 0.0, 1.0) for a in acc]
            el = jnp.where(lane_par, gather_l(acc[1], ilane),
                           gather_l(acc[0], ilane))
            er = jnp.where(lane_par,
                           gather_l(pltpu.roll(acc[1], 64, 1), ilane),
                           gather_l(pltpu.roll(acc[0], 64, 1), ilane))
            ol = jnp.where(lane_par, gather_l(acc[3], ilane),
                           gather_l(acc[2], ilane))
            orr = jnp.where(lane_par,
                            gather_l(pltpu.roll(acc[3], 64, 1), ilane),
                            gather_l(pltpu.roll(acc[2], 64, 1), ilane))
            tlv = jnp.where(sub_par, gather_s(ol, isub_t), gather_s(el, isub_t))
            trv = jnp.where(sub_par, gather_s(orr, isub_t), gather_s(er, isub_t))
            blv = jnp.where(sub_par, gather_s(ol, isub_b), gather_s(el, isub_b))
            brv = jnp.where(sub_par, gather_s(orr, isub_b), gather_s(er, isub_b))
            out_ref[0, pl.ds(2 * r0, 8), pl.ds(2 * c0, 128)] = tlv
            out_ref[0, pl.ds(2 * r0, 8), pl.ds(2 * c0 + 128, 128)] = trv
            out_ref[0, pl.ds(2 * r0 + 8, 8), pl.ds(2 * c0, 128)] = blv
            out_ref[0, pl.ds(2 * r0 + 8, 8), pl.ds(2 * c0 + 128, 128)] = brv
        return carry

    lax.fori_loop(0, H // 8, strip, 0)


@jax.jit
def kernel(img_lr, h_msb, d_msb, b_msb, h_lsb, d_lsb):
    B, C = img_lr.shape[0], img_lr.shape[1]
    tm = _pack_tables((h_msb, d_msb, b_msb), 1.0 / (12.0 * 255.0))
    tl = _pack_tables((h_lsb, d_lsb), 1.0 / (8.0 * 255.0))
    tm = jnp.broadcast_to(tm[:, None, :], (tm.shape[0], 8, 128))
    tl = jnp.broadcast_to(tl[:, None, :], (tl.shape[0], 8, 128))
    img = img_lr.reshape(NPLANES, H, W)
    n_tc = NPLANES - N_SC

    out_sc = _sc_hklut(img[n_tc:],
                       _sc_tables(h_msb, d_msb, b_msb, h_lsb, d_lsb))

    out = pl.pallas_call(
        _hklut_kernel,
        grid=(n_tc,),
        in_specs=[
            pl.BlockSpec((1, H, W), lambda p: (p, 0, 0)),
            pl.BlockSpec((4 * len(MSB_TERMS), 8, 128), lambda p: (0, 0, 0)),
            pl.BlockSpec((4 * len(LSB_TERMS), 8, 128), lambda p: (0, 0, 0)),
        ],
        out_specs=pl.BlockSpec((1, 2 * H, 2 * W), lambda p: (p, 0, 0)),
        out_shape=jax.ShapeDtypeStruct((n_tc, 2 * H, 2 * W), jnp.float32),
        scratch_shapes=[
            pltpu.VMEM((len(_CM_DYS), H, W + 4), jnp.int32),
            pltpu.VMEM((len(_CL_DYS), H, W + 4), jnp.int32),
        ],
        compiler_params=pltpu.CompilerParams(
            dimension_semantics=("arbitrary",),
        ),
    )(img[:n_tc], tm, tl)
    out = jnp.concatenate([out, out_sc], axis=0)
    return out.reshape(B, C, 2 * H, 2 * W)


# speedup vs baseline: 512.6193x; 1.0041x over previous
import dataclasses
import functools

import jax
import jax.numpy as jnp
from jax import lax
from jax.experimental import pallas as pl
from jax.experimental.pallas import tpu as pltpu
from jax.experimental.pallas import tpu_sc as plsc

H = W = 384
NPLANES = 24
N_SC = 20
_PERMS = ((0, 1, 2, 3), (2, 0, 3, 1), (3, 2, 1, 0), (1, 3, 0, 2))


def _delta(k, off):
    dy, dx = off
    return ((dy, dx), (dx, -dy), (-dy, -dx), (-dx, dy))[k]


_MSB_OFFS = ((0, 1), (1, 1), (1, 2))
_LSB_OFFS = ((0, 1), (1, 1))
MSB_TERMS = tuple(_delta(k, off) for off in _MSB_OFFS for k in range(4))
LSB_TERMS = tuple(_delta(k, off) for off in _LSB_OFFS for k in range(4))


def _pack_tables(luts, scale):
    rows = []
    for lut in luts:
        for k in range(4):
            eff = lut[:, list(_PERMS[k])] * scale
            b = eff.astype(jnp.bfloat16)
            u = lax.bitcast_convert_type(b, jnp.uint16).astype(jnp.uint32)
            p01 = (u[:, 0] << 16) | u[:, 1]
            p23 = (u[:, 2] << 16) | u[:, 3]
            rows += [p01[:128], p01[128:], p23[:128], p23[128:]]
    return jnp.stack(rows)


_CM_DYS = (-2, -1, 0, 1, 2)
_CL_DYS = (-1, 0, 1)

_DIR_MAP = {}
for _ti, _d in enumerate(MSB_TERMS):
    _DIR_MAP[_d] = [_ti, None]
for _li, _d in enumerate(LSB_TERMS):
    _DIR_MAP[_d][1] = _li
_DIRS = tuple((dy, dx, v[0], v[1]) for (dy, dx), v in _DIR_MAP.items())


def _sc_tables(h_msb, d_msb, b_msb, h_lsb, d_lsb):
    cols = []
    for luts, scale in (((h_msb, d_msb, b_msb), 1.0 / (12.0 * 255.0)),
                        ((h_lsb, d_lsb), 1.0 / (8.0 * 255.0))):
        for lut in luts:
            for k in range(4):
                eff = lut[:, list(_PERMS[k])] * scale
                b = eff.astype(jnp.bfloat16)
                u = lax.bitcast_convert_type(b, jnp.uint16).astype(jnp.uint32)
                p01 = (u[:, 0] << 16) | u[:, 1]
                p23 = (u[:, 2] << 16) | u[:, 3]
                cols.append(jnp.stack([p01, p23]))
    return lax.bitcast_convert_type(jnp.concatenate(cols, axis=1), jnp.int32)


def _sc_compiler_params():
    cp = pltpu.CompilerParams()
    if "needs_layout_passes" in pltpu.CompilerParams.__dataclass_fields__:
        cp = dataclasses.replace(cp, needs_layout_passes=False)
    return cp


def _sc_hklut(img_sc, tbl):
    n_sc = img_sc.shape[0]
    info = plsc.get_sparse_core_info()
    nc = info.num_cores
    mesh = plsc.VectorSubcoreMesh(core_axis_name="c", subcore_axis_name="s")

    @functools.partial(
        pl.kernel,
        out_type=jax.ShapeDtypeStruct((n_sc, 2 * H, 2 * W), jnp.float32),
        mesh=mesh,
        scratch_types=[
            pltpu.VMEM((2, 24, W), jnp.float32),
            pltpu.VMEM((24, W + 16), jnp.int32),
            pltpu.VMEM((2, 24, 2 * W), jnp.float32),
            pltpu.VMEM((2, 5120), jnp.int32),
            pltpu.SemaphoreType.DMA,
            pltpu.SemaphoreType.DMA,
        ],
        compiler_params=_sc_compiler_params(),
    )
    def k(img_hbm, tbl_hbm, out_hbm, imgc, xic, outc, tblv, sem_in, sem_out):
        wid = lax.axis_index("s") * nc + lax.axis_index("c")
        r0 = wid * 12
        base = pl.multiple_of(jnp.clip(((r0 - 2) // 8) * 8, 0, H - 24), 8)
        off = r0 - base
        iota = lax.iota(jnp.int32, 16)
        iota2 = iota * 2
        zero = jnp.zeros((16,), jnp.int32)
        iota_dx = {dx: iota + (dx + 2) for dx in _CM_DYS}
        cjs = (zero, zero + 1)
        himask = jnp.full((16,), -65536, jnp.int32)
        pltpu.sync_copy(tbl_hbm, tblv)
        hr0 = pl.multiple_of(2 * r0, 8)
        pltpu.async_copy(img_hbm.at[0, pl.ds(base, 24)], imgc.at[0], sem_in)

        @pl.loop(0, n_sc)
        def _plane(p):
            b = p & 1
            pltpu.make_async_copy(img_hbm.at[p, pl.ds(base, 24)],
                                  imgc.at[b], sem_in).wait()

            @pl.when(p + 1 < n_sc)
            def _prefetch():
                pltpu.async_copy(img_hbm.at[p + 1, pl.ds(base, 24)],
                                 imgc.at[1 - b], sem_in)

            @pl.when(p >= 2)
            def _drain_out():
                pltpu.make_async_copy(outc.at[b],
                                      out_hbm.at[p - 2, pl.ds(hr0, 24)],
                                      sem_out).wait()

            @pl.loop(0, 24)
            def _r(r):
                rv = zero + r

                @pl.loop(0, (W + 16) // 16)
                def _c(cc):
                    cv = cc * 16 + iota
                    src = jnp.clip(cv - 2, 0, W - 1)
                    xv = plsc.load_gather(imgc.at[b], [rv, src])
                    xiv = (xv * 255.0).astype(jnp.int32)
                    plsc.store_scatter(xic, [rv, cv], xiv)

            @pl.loop(0, 12)
            def _row(y):
                yy = y + off
                rowvs = {dy: jnp.clip(zero + (yy + dy), 0, 23)
                         for dy in _CM_DYS}
                rowe = zero + 2 * y
                rowo = rowe + 1

                @plsc.parallel_loop(0, W // 16, unroll=4)
                def _x(xx):
                    xb = xx * 16
                    colvs = {dx: xb + iota_dx[dx] for dx in _CM_DYS}
                    rv0 = rowvs[0]
                    xv = plsc.load_gather(imgc.at[b], [rv0, xb + iota])
                    xiv = plsc.load_gather(xic, [rv0, colvs[0]])
                    cmb = (xiv >> 4) << 4
                    clb = (xiv & 15) << 4
                    accs = [xv, xv, xv, xv]
                    for (dy, dx, mt, lt) in _DIRS:
                        nxi = plsc.load_gather(xic, [rowvs[dy], colvs[dx]])
                        cols = []
                        if mt is not None:
                            cols.append(cmb + (nxi >> 4) + (mt * 256))
                        if lt is not None:
                            cols.append(clb + (nxi & 15) + ((12 + lt) * 256))
                        for col in cols:
                            for pr in (0, 1):
                                g = plsc.load_gather(tblv, [cjs[pr], col])
                                accs[2 * pr] = accs[2 * pr] + plsc.bitcast(
                                    g & himask, jnp.float32)
                                accs[2 * pr + 1] = accs[2 * pr + 1] + plsc.bitcast(
                                    g << 16, jnp.float32)
                    accs = [jnp.clip(a, 0.0, 1.0) for a in accs]
                    ce = xb * 2 + iota2
                    plsc.store_scatter(outc.at[b], [rowe, ce], accs[0])
                    plsc.store_scatter(outc.at[b], [rowe, ce + 1], accs[1])
                    plsc.store_scatter(outc.at[b], [rowo, ce], accs[2])
                    plsc.store_scatter(outc.at[b], [rowo, ce + 1], accs[3])

            pltpu.async_copy(outc.at[b], out_hbm.at[p, pl.ds(hr0, 24)],
                             sem_out)

        for i in range(2):
            pltpu.make_async_copy(outc.at[i], out_hbm.at[0, pl.ds(hr0, 24)],
                                  sem_out).wait()

    return k(img_sc, tbl)


def _hklut_kernel(img_ref, tm_ref, tl_ref, out_ref, cmp_ref, clp_ref):
    x_plane = img_ref[0]
    xi = (x_plane * 255.0).astype(jnp.int32)
    cm = xi >> 4
    cl = xi & 15

    def padcols(a):
        return jnp.concatenate([a[:, :1], a[:, :1], a, a[:, -1:], a[:, -1:]],
                               axis=1)

    def rowshift(a, dy):
        if dy == 0:
            return a
        if dy > 0:
            return jnp.concatenate([a[dy:]] + dy * [a[-1:]], axis=0)
        return jnp.concatenate((-dy) * [a[:1]] + [a[:dy]], axis=0)

    for ref, plane, dys in ((cmp_ref, cm, _CM_DYS), (clp_ref, cl, _CL_DYS)):
        for j, dy in enumerate(dys):
            ref[j] = padcols(rowshift(plane, dy))

    def gather_l(tbl, idx):
        return jnp.take_along_axis(tbl, idx, axis=1, mode="promise_in_bounds")

    def gather_s(arr, idx):
        return jnp.take_along_axis(arr, idx, axis=0, mode="promise_in_bounds")

    lane = lax.broadcasted_iota(jnp.int32, (8, 128), 1)
    sub = lax.broadcasted_iota(jnp.int32, (8, 128), 0)
    lane_par = (lane & 1) == 1
    ilane = lane >> 1
    ilane_r = ilane + 64
    sub_par = (sub & 1) == 1
    isub_t = sub >> 1
    isub_b = isub_t + 4

    COLS = (0, 128, 256)

    def strip(i, carry):
        r0 = i * 8
        for c0 in COLS:
            x = img_ref[0, pl.ds(r0, 8), pl.ds(c0, 128)]
            xi_t = (x * 255.0).astype(jnp.int32)
            cm_c = xi_t >> 4
            cl_c = xi_t & 15
            mm = cm_c >= 8
            ml = cl_c >= 8
            cm716 = (cm_c & 7) << 4
            cl716 = (cl_c & 7) << 4
            acc = [x, x, x, x]

            for terms, tref, pad_ref, dys, base16, msk in (
                (MSB_TERMS, tm_ref, cmp_ref, _CM_DYS, cm716, mm),
                (LSB_TERMS, tl_ref, clp_ref, _CL_DYS, cl716, ml),
            ):
                for t, (dy, dx) in enumerate(terms):
                    nb = pad_ref[dys.index(dy), pl.ds(r0, 8),
                                 pl.ds(2 + c0 + dx, 128)]
                    idx7 = base16 | nb
                    for pair in (0, 1):
                        tlo = tref[4 * t + 2 * pair]
                        thi = tref[4 * t + 2 * pair + 1]
                        g = jnp.where(msk, gather_l(thi, idx7),
                                      gather_l(tlo, idx7))
                        acc[2 * pair] += lax.bitcast_convert_type(
                            g & jnp.uint32(0xFFFF0000), jnp.float32)
                        acc[2 * pair + 1] += lax.bitcast_convert_type(
                            g << 16, jnp.float32)

            acc = [jnp.clip(a, 0.0, 1.0) for a in acc]
            el = jnp.where(lane_par, gather_l(acc[1], ilane),
                           gather_l(acc[0], ilane))
            er = jnp.where(lane_par,
                           gather_l(pltpu.roll(acc[1], 64, 1), ilane),
                           gather_l(pltpu.roll(acc[0], 64, 1), ilane))
            ol = jnp.where(lane_par, gather_l(acc[3], ilane),
                           gather_l(acc[2], ilane))
            orr = jnp.where(lane_par,
                            gather_l(pltpu.roll(acc[3], 64, 1), ilane),
                            gather_l(pltpu.roll(acc[2], 64, 1), ilane))
            tlv = jnp.where(sub_par, gather_s(ol, isub_t), gather_s(el, isub_t))
            trv = jnp.where(sub_par, gather_s(orr, isub_t), gather_s(er, isub_t))
            blv = jnp.where(sub_par, gather_s(ol, isub_b), gather_s(el, isub_b))
            brv = jnp.where(sub_par, gather_s(orr, isub_b), gather_s(er, isub_b))
            out_ref[0, pl.ds(2 * r0, 8), pl.ds(2 * c0, 128)] = tlv
            out_ref[0, pl.ds(2 * r0, 8), pl.ds(2 * c0 + 128, 128)] = trv
            out_ref[0, pl.ds(2 * r0 + 8, 8), pl.ds(2 * c0, 128)] = blv
            out_ref[0, pl.ds(2 * r0 + 8, 8), pl.ds(2 * c0 + 128, 128)] = brv
        return carry

    lax.fori_loop(0, H // 8, strip, 0)


@jax.jit
def kernel(img_lr, h_msb, d_msb, b_msb, h_lsb, d_lsb):
    B, C = img_lr.shape[0], img_lr.shape[1]
    tm = _pack_tables((h_msb, d_msb, b_msb), 1.0 / (12.0 * 255.0))
    tl = _pack_tables((h_lsb, d_lsb), 1.0 / (8.0 * 255.0))
    tm = jnp.broadcast_to(tm[:, None, :], (tm.shape[0], 8, 128))
    tl = jnp.broadcast_to(tl[:, None, :], (tl.shape[0], 8, 128))
    img = img_lr.reshape(NPLANES, H, W)
    n_tc = NPLANES - N_SC

    out_sc = _sc_hklut(img[n_tc:],
                       _sc_tables(h_msb, d_msb, b_msb, h_lsb, d_lsb))

    out = pl.pallas_call(
        _hklut_kernel,
        grid=(n_tc,),
        in_specs=[
            pl.BlockSpec((1, H, W), lambda p: (p, 0, 0)),
            pl.BlockSpec((4 * len(MSB_TERMS), 8, 128), lambda p: (0, 0, 0)),
            pl.BlockSpec((4 * len(LSB_TERMS), 8, 128), lambda p: (0, 0, 0)),
        ],
        out_specs=pl.BlockSpec((1, 2 * H, 2 * W), lambda p: (p, 0, 0)),
        out_shape=jax.ShapeDtypeStruct((n_tc, 2 * H, 2 * W), jnp.float32),
        scratch_shapes=[
            pltpu.VMEM((len(_CM_DYS), H, W + 4), jnp.int32),
            pltpu.VMEM((len(_CL_DYS), H, W + 4), jnp.int32),
        ],
        compiler_params=pltpu.CompilerParams(
            dimension_semantics=("arbitrary",),
        ),
    )(img[:n_tc], tm, tl)
    out = jnp.concatenate([out, out_sc], axis=0)
    return out.reshape(B, C, 2 * H, 2 * W)
